# baseline (speedup 1.0000x reference)
import functools

import jax
import jax.numpy as jnp
from jax import lax
from jax.experimental import pallas as pl
from jax.experimental.pallas import tpu as pltpu
from jax.experimental.pallas import tpu_sc as plsc

S = 2048
H = 1024
NH = 16
NKV = 8
HD = 64
E = 8
K = 2
F = 1024
EPS = 1e-5
THETA = 150000.0
ALPHA = 1.702
LIMIT = 7.0
HALF = HD // 2

BQ = 512
BK = 512
NQ = S // BQ
NJ = S // BK
BR = 256
NR = S // BR
TM = 128
NT = (S * K) // TM + E
NPAD = NT * TM


def _qkv_body(x_ref, ln_ref, w_ref, cos_ref, sin_ref, q_ref, k_ref, v_ref):
    x = x_ref[...]
    var = jnp.mean(x * x, axis=-1, keepdims=True)
    xn = (x * lax.rsqrt(var + EPS)) * ln_ref[...]
    qkv = jnp.dot(xn, w_ref[...], preferred_element_type=jnp.float32)
    cos = cos_ref[...]
    sin = sin_ref[...]

    def rope_head(b):
        x1 = qkv[:, b:b + HALF]
        x2 = qkv[:, b + HALF:b + HD]
        return jnp.concatenate([x1 * cos - x2 * sin, x1 * sin + x2 * cos],
                               axis=1)

    for h in range(NH):
        q_ref[h] = rope_head(h * HD)
    for h in range(NKV):
        k_ref[h] = rope_head((NH + h) * HD)
        v_ref[h] = qkv[:, (NH + NKV + h) * HD:(NH + NKV + h + 1) * HD]


def _qkv_call(x, ln1_w, wqkv, cos, sin):
    return pl.pallas_call(
        _qkv_body,
        grid=(NR,),
        in_specs=[
            pl.BlockSpec((BR, H), lambda i: (i, 0)),
            pl.BlockSpec((1, H), lambda i: (0, 0)),
            pl.BlockSpec((H, (NH + 2 * NKV) * HD), lambda i: (0, 0)),
            pl.BlockSpec((BR, HALF), lambda i: (i, 0)),
            pl.BlockSpec((BR, HALF), lambda i: (i, 0)),
        ],
        out_specs=[
            pl.BlockSpec((NH, BR, HD), lambda i: (0, i, 0)),
            pl.BlockSpec((NKV, BR, HD), lambda i: (0, i, 0)),
            pl.BlockSpec((NKV, BR, HD), lambda i: (0, i, 0)),
        ],
        out_shape=[
            jax.ShapeDtypeStruct((NH, S, HD), jnp.float32),
            jax.ShapeDtypeStruct((NKV, S, HD), jnp.float32),
            jax.ShapeDtypeStruct((NKV, S, HD), jnp.float32),
        ],
    )(x, ln1_w, wqkv, cos, sin)


def _attn_body(sink_ref, q_ref, k_ref, v_ref, o_ref, m_ref, l_ref, acc_ref):
    h = pl.program_id(0)
    i = pl.program_id(1)
    j = pl.program_id(2)

    @pl.when(j == 0)
    def _init():
        m_ref[...] = jnp.full((BQ, 1), -1e30, jnp.float32)
        l_ref[...] = jnp.zeros((BQ, 1), jnp.float32)
        acc_ref[...] = jnp.zeros((BQ, HD), jnp.float32)

    @pl.when(j * BK < (i + 1) * BQ)
    def _compute():
        q = q_ref[0]
        k = k_ref[0]
        s = lax.dot_general(q, k, (((1,), (1,)), ((), ())),
                            preferred_element_type=jnp.float32)
        s = s * (HD ** -0.5)
        rows = i * BQ + lax.broadcasted_iota(jnp.int32, (BQ, BK), 0)
        cols = j * BK + lax.broadcasted_iota(jnp.int32, (BQ, BK), 1)
        s = jnp.where(rows >= cols, s, -1e30)
        m_prev = m_ref[...]
        m_cur = jnp.maximum(m_prev, jnp.max(s, axis=1, keepdims=True))
        alpha = jnp.exp(m_prev - m_cur)
        p = jnp.exp(s - m_cur)
        l_ref[...] = l_ref[...] * alpha + jnp.sum(p, axis=1, keepdims=True)
        acc_ref[...] = acc_ref[...] * alpha + jnp.dot(
            p, v_ref[0], preferred_element_type=jnp.float32)
        m_ref[...] = m_cur

    @pl.when(j == NJ - 1)
    def _fin():
        sink = sink_ref[h]
        l = l_ref[...] + jnp.exp(sink - m_ref[...])
        o_ref[0] = acc_ref[...] / l


def _attn_call(q, k, v, sinks):
    return pl.pallas_call(
        _attn_body,
        grid=(NH, NQ, NJ),
        in_specs=[
            pl.BlockSpec(memory_space=pltpu.SMEM),
            pl.BlockSpec((1, BQ, HD), lambda h, i, j: (h, i, 0)),
            pl.BlockSpec((1, BK, HD),
                         lambda h, i, j: (h // 2,
                                          jnp.minimum(j, (i * BQ + BQ - 1) // BK),
                                          0)),
            pl.BlockSpec((1, BK, HD),
                         lambda h, i, j: (h // 2,
                                          jnp.minimum(j, (i * BQ + BQ - 1) // BK),
                                          0)),
        ],
        out_specs=pl.BlockSpec((1, BQ, HD), lambda h, i, j: (h, i, 0)),
        out_shape=jax.ShapeDtypeStruct((NH, S, HD), jnp.float32),
        scratch_shapes=[
            pltpu.VMEM((BQ, 1), jnp.float32),
            pltpu.VMEM((BQ, 1), jnp.float32),
            pltpu.VMEM((BQ, HD), jnp.float32),
        ],
    )(sinks, q, k, v)


def _post_body(a_ref, wo_ref, hs_ref, ln_ref, rw_ref, rb_ref,
               hid_ref, h2_ref, oh1_ref, oh2_ref, w_ref):
    att = jnp.dot(a_ref[0], wo_ref[0], preferred_element_type=jnp.float32)
    for h in range(1, NH):
        att = att + jnp.dot(a_ref[h], wo_ref[h],
                            preferred_element_type=jnp.float32)
    hid = att + hs_ref[...]
    hid_ref[...] = hid
    var = jnp.mean(hid * hid, axis=-1, keepdims=True)
    h2 = (hid * lax.rsqrt(var + EPS)) * ln_ref[...]
    h2_ref[...] = h2
    lg = jnp.dot(h2, rw_ref[...],
                 preferred_element_type=jnp.float32) + rb_ref[...]
    iota_e = lax.broadcasted_iota(jnp.int32, (BR, E), 1)
    m1 = jnp.max(lg, axis=1, keepdims=True)
    i1 = jnp.min(jnp.where(lg >= m1, iota_e, E), axis=1, keepdims=True)
    oh1 = iota_e == i1
    lg2 = jnp.where(oh1, -1e30, lg)
    m2 = jnp.max(lg2, axis=1, keepdims=True)
    i2 = jnp.min(jnp.where(lg2 >= m2, iota_e, E), axis=1, keepdims=True)
    oh2 = iota_e == i2
    oh1_ref[...] = oh1.astype(jnp.int32)
    oh2_ref[...] = oh2.astype(jnp.int32)
    w1 = jax.nn.sigmoid(m1 - m2)
    w_ref[...] = jnp.concatenate([w1, 1.0 - w1], axis=1)


def _post_call(attn, wo, hs, ln2_w, router_w, router_b):
    return pl.pallas_call(
        _post_body,
        grid=(NR,),
        in_specs=[
            pl.BlockSpec((NH, BR, HD), lambda i: (0, i, 0)),
            pl.BlockSpec((NH, HD, H), lambda i: (0, 0, 0)),
            pl.BlockSpec((BR, H), lambda i: (i, 0)),
            pl.BlockSpec((1, H), lambda i: (0, 0)),
            pl.BlockSpec((H, E), lambda i: (0, 0)),
            pl.BlockSpec((1, E), lambda i: (0, 0)),
        ],
        out_specs=[
            pl.BlockSpec((BR, H), lambda i: (i, 0)),
            pl.BlockSpec((BR, H), lambda i: (i, 0)),
            pl.BlockSpec((BR, E), lambda i: (i, 0)),
            pl.BlockSpec((BR, E), lambda i: (i, 0)),
            pl.BlockSpec((BR, K), lambda i: (i, 0)),
        ],
        out_shape=[
            jax.ShapeDtypeStruct((S, H), jnp.float32),
            jax.ShapeDtypeStruct((S, H), jnp.float32),
            jax.ShapeDtypeStruct((S, E), jnp.int32),
            jax.ShapeDtypeStruct((S, E), jnp.int32),
            jax.ShapeDtypeStruct((S, K), jnp.float32),
        ],
    )(attn, wo, hs, ln2_w, router_w, router_b)


@functools.lru_cache(maxsize=None)
def _make_sc_gather(v_rows, d, b_rows, chunk):
    info = plsc.get_sparse_core_info()
    nc, ns = info.num_cores, info.num_subcores
    nw = nc * ns
    b_per_w = b_rows // nw
    nch = b_per_w // chunk
    mesh = plsc.VectorSubcoreMesh(core_axis_name="c", subcore_axis_name="s")

    @functools.partial(
        pl.kernel,
        mesh=mesh,
        out_type=jax.ShapeDtypeStruct((b_rows, d), jnp.float32),
        scratch_types=[
            pltpu.VMEM((chunk,), jnp.int32),
            pltpu.VMEM((chunk, d), jnp.float32),
            pltpu.SemaphoreType.DMA,
        ],
    )
    def gather(table_hbm, idx_hbm, out_hbm, idx_v, rows_v, sem):
        wid = lax.axis_index("s") * nc + lax.axis_index("c")
        base = wid * b_per_w

        def body(c, carry):
            off = base + c * chunk
            pltpu.sync_copy(idx_hbm.at[pl.ds(off, chunk)], idx_v)
            pltpu.async_copy(table_hbm.at[idx_v], rows_v, sem).wait()
            pltpu.sync_copy(rows_v, out_hbm.at[pl.ds(off, chunk)])
            return carry

        lax.fori_loop(0, nch, body, 0)

    return gather


def _gmm_body(eid_ref, act_ref, xs_ref, guw_ref, gub_ref,
              dw_ref, db_ref, out_ref, dz_ref):
    t = pl.program_id(0)

    @pl.when(act_ref[t] == 1)
    def _():
        changed = (t == 0) | (eid_ref[t] != eid_ref[jnp.maximum(t - 1, 0)])

        @pl.when(changed)
        def _rebuild():
            dz_ref[...] = jnp.repeat(dw_ref[0], 2, axis=0)

        x = xs_ref[...]
        gu = jnp.dot(x, guw_ref[0],
                     preferred_element_type=jnp.float32) + gub_ref[0]
        g = jnp.minimum(gu, LIMIT)
        glu = g * jax.nn.sigmoid(g * ALPHA)
        u1 = jnp.clip(gu, -LIMIT, LIMIT) + 1.0
        lane = lax.broadcasted_iota(jnp.int32, (TM, 2 * F), 1)
        hh = jnp.where(lane % 2 == 0, glu * pltpu.roll(u1, 2 * F - 1, 1),
                       0.0)
        out_ref[...] = jnp.dot(hh, dz_ref[...],
                               preferred_element_type=jnp.float32) + db_ref[0]


def _gmm_call(tile_eid, tile_act, xs_pad, gate_up_w, gate_up_b,
              down_w, down_b):
    grid_spec = pltpu.PrefetchScalarGridSpec(
        num_scalar_prefetch=2,
        grid=(NT,),
        in_specs=[
            pl.BlockSpec((TM, H), lambda t, eid, act: (t, 0)),
            pl.BlockSpec((1, H, 2 * F), lambda t, eid, act: (eid[t], 0, 0)),
            pl.BlockSpec((1, 1, 2 * F), lambda t, eid, act: (eid[t], 0, 0)),
            pl.BlockSpec((1, F, H), lambda t, eid, act: (eid[t], 0, 0)),
            pl.BlockSpec((1, 1, H), lambda t, eid, act: (eid[t], 0, 0)),
        ],
        out_specs=pl.BlockSpec((TM, H), lambda t, eid, act: (t, 0)),
        scratch_shapes=[pltpu.VMEM((2 * F, H), jnp.float32)],
    )
    return pl.pallas_call(
        _gmm_body,
        grid_spec=grid_spec,
        out_shape=jax.ShapeDtypeStruct((NPAD, H), jnp.float32),
    )(tile_eid, tile_act, xs_pad, gate_up_w, gate_up_b, down_w, down_b)


def _comb_body(hid_ref, sr_ref, w_ref, out_ref):
    srv = sr_ref[...]
    w = w_ref[...]
    out_ref[...] = (hid_ref[...] + w[:, 0:1] * srv[:, :H]
                    + w[:, 1:2] * srv[:, H:])


def _comb_call(hidden, slot_rows, w):
    return pl.pallas_call(
        _comb_body,
        grid=(NR,),
        in_specs=[
            pl.BlockSpec((BR, H), lambda i: (i, 0)),
            pl.BlockSpec((BR, 2 * H), lambda i: (i, 0)),
            pl.BlockSpec((BR, K), lambda i: (i, 0)),
        ],
        out_specs=pl.BlockSpec((BR, H), lambda i: (i, 0)),
        out_shape=jax.ShapeDtypeStruct((S, H), jnp.float32),
    )(hidden, slot_rows, w)


def _routing(oh1, oh2):
    oh = jnp.stack([oh1, oh2], axis=1).reshape(S * K, E)
    incl = jnp.cumsum(oh, axis=0)
    counts = incl[-1]
    rank = jnp.sum((incl - oh) * oh, axis=1)
    padded = ((counts + TM - 1) // TM) * TM
    ends_pad = jnp.cumsum(padded)
    padoff = ends_pad - padded
    pos_flat = (jnp.sum(oh * padoff[None, :], axis=1) + rank).astype(jnp.int32)
    slots = jnp.arange(S * K, dtype=jnp.int32)
    src_tok = jnp.zeros((NPAD,), jnp.int32).at[pos_flat].set(slots // K)
    tstart = jnp.arange(NT, dtype=jnp.int32)[:, None] * TM
    tile_eid = jnp.sum((tstart >= ends_pad[None, :]).astype(jnp.int32), axis=1)
    tile_eid_c = jnp.minimum(tile_eid, E - 1)
    toh = (tile_eid_c[:, None] == jnp.arange(E, dtype=jnp.int32)[None, :]
           ).astype(jnp.int32)
    tile_act = (((tstart[:, 0] - jnp.sum(toh * padoff[None, :], axis=1))
                 < jnp.sum(toh * counts[None, :], axis=1))
                & (tile_eid < E)).astype(jnp.int32)
    return src_tok, pos_flat, tile_eid_c, tile_act


def kernel(hidden_states, positions, ln1_w, wq, wk, wv, wo, sinks, ln2_w,
           router_w, router_b, gate_up_w, gate_up_b, down_w, down_b):
    x = hidden_states
    wqkv = jnp.concatenate([wq, wk, wv], axis=1)
    inv = 1.0 / (THETA ** (jnp.arange(HALF, dtype=jnp.float32) / HALF))
    ang = positions.astype(jnp.float32)[:, None] * inv[None, :]
    cos = jnp.cos(ang)
    sin = jnp.sin(ang)

    q, k, v = _qkv_call(x, ln1_w.reshape(1, H), wqkv, cos, sin)
    attn = _attn_call(q, k, v, sinks)
    hidden, h2, oh1, oh2, w = _post_call(attn, wo.reshape(NH, HD, H), x,
                                         ln2_w.reshape(1, H),
                                         router_w, router_b.reshape(1, E))

    src_tok, pos_flat, tile_eid, tile_act = _routing(oh1, oh2)

    xs_pad = _make_sc_gather(S, H, NPAD, 32)(h2, src_tok)
    rows = _gmm_call(tile_eid, tile_act, xs_pad, gate_up_w,
                     gate_up_b.reshape(E, 1, 2 * F), down_w,
                     down_b.reshape(E, 1, H))
    slot_rows = _make_sc_gather(NPAD, H, S * K, 32)(rows, pos_flat)
    return _comb_call(hidden, slot_rows.reshape(S, 2 * H), w)


# speedup vs baseline: 1.2551x; 1.2551x over previous
import functools

import jax
import jax.numpy as jnp
from jax import lax
from jax.experimental import pallas as pl
from jax.experimental.pallas import tpu as pltpu
from jax.experimental.pallas import tpu_sc as plsc

S = 2048
H = 1024
NH = 16
NKV = 8
HD = 64
E = 8
K = 2
F = 1024
EPS = 1e-5
THETA = 150000.0
ALPHA = 1.702
LIMIT = 7.0
HALF = HD // 2

BQ = 1024
BK = 1024
NQ = S // BQ
NJ = S // BK
BR = 256
NR = S // BR
TM = 128
NT = (S * K) // TM + E
NPAD = NT * TM


def _qkv_body(x_ref, ln_ref, w_ref, cos_ref, sin_ref, q_ref, k_ref, v_ref):
    x = x_ref[...]
    var = jnp.mean(x * x, axis=-1, keepdims=True)
    xn = (x * lax.rsqrt(var + EPS)) * ln_ref[...]
    qkv = jnp.dot(xn, w_ref[...], preferred_element_type=jnp.float32)
    cos = cos_ref[...]
    sin = sin_ref[...]

    def rope_head(b):
        x1 = qkv[:, b:b + HALF]
        x2 = qkv[:, b + HALF:b + HD]
        return jnp.concatenate([x1 * cos - x2 * sin, x1 * sin + x2 * cos],
                               axis=1)

    for h in range(NH):
        q_ref[h] = rope_head(h * HD)
    for h in range(NKV):
        k_ref[h] = rope_head((NH + h) * HD)
        v_ref[h] = qkv[:, (NH + NKV + h) * HD:(NH + NKV + h + 1) * HD]


def _qkv_call(x, ln1_w, wqkv, cos, sin):
    return pl.pallas_call(
        _qkv_body,
        grid=(NR,),
        in_specs=[
            pl.BlockSpec((BR, H), lambda i: (i, 0)),
            pl.BlockSpec((1, H), lambda i: (0, 0)),
            pl.BlockSpec((H, (NH + 2 * NKV) * HD), lambda i: (0, 0)),
            pl.BlockSpec((BR, HALF), lambda i: (i, 0)),
            pl.BlockSpec((BR, HALF), lambda i: (i, 0)),
        ],
        out_specs=[
            pl.BlockSpec((NH, BR, HD), lambda i: (0, i, 0)),
            pl.BlockSpec((NKV, BR, HD), lambda i: (0, i, 0)),
            pl.BlockSpec((NKV, BR, HD), lambda i: (0, i, 0)),
        ],
        out_shape=[
            jax.ShapeDtypeStruct((NH, S, HD), jnp.float32),
            jax.ShapeDtypeStruct((NKV, S, HD), jnp.float32),
            jax.ShapeDtypeStruct((NKV, S, HD), jnp.float32),
        ],
    )(x, ln1_w, wqkv, cos, sin)


def _attn_body(sink_ref, q_ref, k_ref, v_ref, o_ref, m_ref, l_ref, acc_ref):
    h = pl.program_id(0)
    i = pl.program_id(1)
    j = pl.program_id(2)

    @pl.when(j == 0)
    def _init():
        m_ref[...] = jnp.full((BQ, 1), -1e30, jnp.float32)
        l_ref[...] = jnp.zeros((BQ, 1), jnp.float32)
        acc_ref[...] = jnp.zeros((BQ, HD), jnp.float32)

    @pl.when(j * BK < (i + 1) * BQ)
    def _compute():
        q = q_ref[0]
        k = k_ref[0]
        s = lax.dot_general(q, k, (((1,), (1,)), ((), ())),
                            preferred_element_type=jnp.float32)
        s = s * (HD ** -0.5)
        rows = i * BQ + lax.broadcasted_iota(jnp.int32, (BQ, BK), 0)
        cols = j * BK + lax.broadcasted_iota(jnp.int32, (BQ, BK), 1)
        s = jnp.where(rows >= cols, s, -1e30)
        m_prev = m_ref[...]
        m_cur = jnp.maximum(m_prev, jnp.max(s, axis=1, keepdims=True))
        alpha = jnp.exp(m_prev - m_cur)
        p = jnp.exp(s - m_cur)
        l_ref[...] = l_ref[...] * alpha + jnp.sum(p, axis=1, keepdims=True)
        acc_ref[...] = acc_ref[...] * alpha + jnp.dot(
            p, v_ref[0], preferred_element_type=jnp.float32)
        m_ref[...] = m_cur

    @pl.when(j == NJ - 1)
    def _fin():
        sink = sink_ref[h]
        l = l_ref[...] + jnp.exp(sink - m_ref[...])
        o_ref[0] = acc_ref[...] / l


def _attn_call(q, k, v, sinks):
    return pl.pallas_call(
        _attn_body,
        grid=(NH, NQ, NJ),
        in_specs=[
            pl.BlockSpec(memory_space=pltpu.SMEM),
            pl.BlockSpec((1, BQ, HD), lambda h, i, j: (h, i, 0)),
            pl.BlockSpec((1, BK, HD),
                         lambda h, i, j: (h // 2,
                                          jnp.minimum(j, (i * BQ + BQ - 1) // BK),
                                          0)),
            pl.BlockSpec((1, BK, HD),
                         lambda h, i, j: (h // 2,
                                          jnp.minimum(j, (i * BQ + BQ - 1) // BK),
                                          0)),
        ],
        out_specs=pl.BlockSpec((1, BQ, HD), lambda h, i, j: (h, i, 0)),
        out_shape=jax.ShapeDtypeStruct((NH, S, HD), jnp.float32),
        scratch_shapes=[
            pltpu.VMEM((BQ, 1), jnp.float32),
            pltpu.VMEM((BQ, 1), jnp.float32),
            pltpu.VMEM((BQ, HD), jnp.float32),
        ],
    )(sinks, q, k, v)


def _post_body(a_ref, wo_ref, hs_ref, ln_ref, rw_ref, rb_ref,
               hid_ref, h2_ref, oh1_ref, oh2_ref, w_ref):
    att = jnp.dot(a_ref[0], wo_ref[0], preferred_element_type=jnp.float32)
    for h in range(1, NH):
        att = att + jnp.dot(a_ref[h], wo_ref[h],
                            preferred_element_type=jnp.float32)
    hid = att + hs_ref[...]
    hid_ref[...] = hid
    var = jnp.mean(hid * hid, axis=-1, keepdims=True)
    h2 = (hid * lax.rsqrt(var + EPS)) * ln_ref[...]
    h2_ref[...] = h2
    lg = jnp.dot(h2, rw_ref[...],
                 preferred_element_type=jnp.float32) + rb_ref[...]
    iota_e = lax.broadcasted_iota(jnp.int32, (BR, E), 1)
    m1 = jnp.max(lg, axis=1, keepdims=True)
    i1 = jnp.min(jnp.where(lg >= m1, iota_e, E), axis=1, keepdims=True)
    oh1 = iota_e == i1
    lg2 = jnp.where(oh1, -1e30, lg)
    m2 = jnp.max(lg2, axis=1, keepdims=True)
    i2 = jnp.min(jnp.where(lg2 >= m2, iota_e, E), axis=1, keepdims=True)
    oh2 = iota_e == i2
    oh1_ref[...] = oh1.astype(jnp.int32)
    oh2_ref[...] = oh2.astype(jnp.int32)
    w1 = jax.nn.sigmoid(m1 - m2)
    w_ref[...] = jnp.concatenate([w1, 1.0 - w1], axis=1)


def _post_call(attn, wo, hs, ln2_w, router_w, router_b):
    return pl.pallas_call(
        _post_body,
        grid=(NR,),
        in_specs=[
            pl.BlockSpec((NH, BR, HD), lambda i: (0, i, 0)),
            pl.BlockSpec((NH, HD, H), lambda i: (0, 0, 0)),
            pl.BlockSpec((BR, H), lambda i: (i, 0)),
            pl.BlockSpec((1, H), lambda i: (0, 0)),
            pl.BlockSpec((H, E), lambda i: (0, 0)),
            pl.BlockSpec((1, E), lambda i: (0, 0)),
        ],
        out_specs=[
            pl.BlockSpec((BR, H), lambda i: (i, 0)),
            pl.BlockSpec((BR, H), lambda i: (i, 0)),
            pl.BlockSpec((BR, E), lambda i: (i, 0)),
            pl.BlockSpec((BR, E), lambda i: (i, 0)),
            pl.BlockSpec((BR, K), lambda i: (i, 0)),
        ],
        out_shape=[
            jax.ShapeDtypeStruct((S, H), jnp.float32),
            jax.ShapeDtypeStruct((S, H), jnp.float32),
            jax.ShapeDtypeStruct((S, E), jnp.int32),
            jax.ShapeDtypeStruct((S, E), jnp.int32),
            jax.ShapeDtypeStruct((S, K), jnp.float32),
        ],
    )(attn, wo, hs, ln2_w, router_w, router_b)


@functools.lru_cache(maxsize=None)
def _make_sc_gather(v_rows, d, b_rows, chunk):
    info = plsc.get_sparse_core_info()
    nc, ns = info.num_cores, info.num_subcores
    nw = nc * ns
    b_per_w = b_rows // nw
    nch = b_per_w // chunk
    mesh = plsc.VectorSubcoreMesh(core_axis_name="c", subcore_axis_name="s")

    @functools.partial(
        pl.kernel,
        mesh=mesh,
        out_type=jax.ShapeDtypeStruct((b_rows, d), jnp.float32),
        scratch_types=[
            pltpu.VMEM((chunk,), jnp.int32),
            pltpu.VMEM((chunk, d), jnp.float32),
            pltpu.SemaphoreType.DMA,
        ],
    )
    def gather(table_hbm, idx_hbm, out_hbm, idx_v, rows_v, sem):
        wid = lax.axis_index("s") * nc + lax.axis_index("c")
        base = wid * b_per_w

        def body(c, carry):
            off = base + c * chunk
            pltpu.sync_copy(idx_hbm.at[pl.ds(off, chunk)], idx_v)
            pltpu.async_copy(table_hbm.at[idx_v], rows_v, sem).wait()
            pltpu.sync_copy(rows_v, out_hbm.at[pl.ds(off, chunk)])
            return carry

        lax.fori_loop(0, nch, body, 0)

    return gather


def _gmm_body(eid_ref, act_ref, xs_ref, guw_ref, gub_ref,
              dw_ref, db_ref, out_ref, dz_ref):
    t = pl.program_id(0)

    @pl.when(act_ref[t] == 1)
    def _():
        changed = (t == 0) | (eid_ref[t] != eid_ref[jnp.maximum(t - 1, 0)])

        @pl.when(changed)
        def _rebuild():
            dz_ref[...] = jnp.repeat(dw_ref[0], 2, axis=0)

        x = xs_ref[...]
        gu = jnp.dot(x, guw_ref[0],
                     preferred_element_type=jnp.float32) + gub_ref[0]
        g = jnp.minimum(gu, LIMIT)
        glu = g * jax.nn.sigmoid(g * ALPHA)
        u1 = jnp.clip(gu, -LIMIT, LIMIT) + 1.0
        lane = lax.broadcasted_iota(jnp.int32, (TM, 2 * F), 1)
        hh = jnp.where(lane % 2 == 0, glu * pltpu.roll(u1, 2 * F - 1, 1),
                       0.0)
        out_ref[...] = jnp.dot(hh, dz_ref[...],
                               preferred_element_type=jnp.float32) + db_ref[0]


def _gmm_call(tile_eid, tile_act, xs_pad, gate_up_w, gate_up_b,
              down_w, down_b):
    grid_spec = pltpu.PrefetchScalarGridSpec(
        num_scalar_prefetch=2,
        grid=(NT,),
        in_specs=[
            pl.BlockSpec((TM, H), lambda t, eid, act: (t, 0)),
            pl.BlockSpec((1, H, 2 * F), lambda t, eid, act: (eid[t], 0, 0)),
            pl.BlockSpec((1, 1, 2 * F), lambda t, eid, act: (eid[t], 0, 0)),
            pl.BlockSpec((1, F, H), lambda t, eid, act: (eid[t], 0, 0)),
            pl.BlockSpec((1, 1, H), lambda t, eid, act: (eid[t], 0, 0)),
        ],
        out_specs=pl.BlockSpec((TM, H), lambda t, eid, act: (t, 0)),
        scratch_shapes=[pltpu.VMEM((2 * F, H), jnp.float32)],
    )
    return pl.pallas_call(
        _gmm_body,
        grid_spec=grid_spec,
        out_shape=jax.ShapeDtypeStruct((NPAD, H), jnp.float32),
    )(tile_eid, tile_act, xs_pad, gate_up_w, gate_up_b, down_w, down_b)


def _comb_body(hid_ref, sr_ref, w_ref, out_ref):
    srv = sr_ref[...]
    w = w_ref[...]
    out_ref[...] = (hid_ref[...] + w[:, 0:1] * srv[:, :H]
                    + w[:, 1:2] * srv[:, H:])


def _comb_call(hidden, slot_rows, w):
    return pl.pallas_call(
        _comb_body,
        grid=(NR,),
        in_specs=[
            pl.BlockSpec((BR, H), lambda i: (i, 0)),
            pl.BlockSpec((BR, 2 * H), lambda i: (i, 0)),
            pl.BlockSpec((BR, K), lambda i: (i, 0)),
        ],
        out_specs=pl.BlockSpec((BR, H), lambda i: (i, 0)),
        out_shape=jax.ShapeDtypeStruct((S, H), jnp.float32),
    )(hidden, slot_rows, w)


def _routing(oh1, oh2):
    oh = jnp.stack([oh1, oh2], axis=1).reshape(S * K, E)
    incl = jnp.cumsum(oh, axis=0)
    counts = incl[-1]
    rank = jnp.sum((incl - oh) * oh, axis=1)
    padded = ((counts + TM - 1) // TM) * TM
    ends_pad = jnp.cumsum(padded)
    padoff = ends_pad - padded
    pos_flat = (jnp.sum(oh * padoff[None, :], axis=1) + rank).astype(jnp.int32)
    slots = jnp.arange(S * K, dtype=jnp.int32)
    src_tok = jnp.zeros((NPAD,), jnp.int32).at[pos_flat].set(slots // K)
    tstart = jnp.arange(NT, dtype=jnp.int32)[:, None] * TM
    tile_eid = jnp.sum((tstart >= ends_pad[None, :]).astype(jnp.int32), axis=1)
    tile_eid_c = jnp.minimum(tile_eid, E - 1)
    toh = (tile_eid_c[:, None] == jnp.arange(E, dtype=jnp.int32)[None, :]
           ).astype(jnp.int32)
    tile_act = (((tstart[:, 0] - jnp.sum(toh * padoff[None, :], axis=1))
                 < jnp.sum(toh * counts[None, :], axis=1))
                & (tile_eid < E)).astype(jnp.int32)
    return src_tok, pos_flat, tile_eid_c, tile_act


def kernel(hidden_states, positions, ln1_w, wq, wk, wv, wo, sinks, ln2_w,
           router_w, router_b, gate_up_w, gate_up_b, down_w, down_b):
    x = hidden_states
    wqkv = jnp.concatenate([wq, wk, wv], axis=1)
    inv = 1.0 / (THETA ** (jnp.arange(HALF, dtype=jnp.float32) / HALF))
    ang = positions.astype(jnp.float32)[:, None] * inv[None, :]
    cos = jnp.cos(ang)
    sin = jnp.sin(ang)

    q, k, v = _qkv_call(x, ln1_w.reshape(1, H), wqkv, cos, sin)
    attn = _attn_call(q, k, v, sinks)
    hidden, h2, oh1, oh2, w = _post_call(attn, wo.reshape(NH, HD, H), x,
                                         ln2_w.reshape(1, H),
                                         router_w, router_b.reshape(1, E))

    src_tok, pos_flat, tile_eid, tile_act = _routing(oh1, oh2)

    xs_pad = _make_sc_gather(S, H, NPAD, 32)(h2, src_tok)
    rows = _gmm_call(tile_eid, tile_act, xs_pad, gate_up_w,
                     gate_up_b.reshape(E, 1, 2 * F), down_w,
                     down_b.reshape(E, 1, H))
    slot_rows = _make_sc_gather(NPAD, H, S * K, 32)(rows, pos_flat)
    return _comb_call(hidden, slot_rows.reshape(S, 2 * H), w)


# speedup vs baseline: 1.4364x; 1.1445x over previous
import functools

import jax
import jax.numpy as jnp
from jax import lax
from jax.experimental import pallas as pl
from jax.experimental.pallas import tpu as pltpu
from jax.experimental.pallas import tpu_sc as plsc

S = 2048
H = 1024
NH = 16
NKV = 8
HD = 64
E = 8
K = 2
F = 1024
EPS = 1e-5
THETA = 150000.0
ALPHA = 1.702
LIMIT = 7.0
HALF = HD // 2

BQ = 1024
BK = 1024
NQ = S // BQ
NJ = S // BK
BR = 256
NR = S // BR
TM = 128
NT = (S * K) // TM + E
NPAD = NT * TM


def _qkv_body(x_ref, ln_ref, w_ref, cos_ref, sin_ref, q_ref, k_ref, v_ref):
    x = x_ref[...]
    var = jnp.mean(x * x, axis=-1, keepdims=True)
    xn = (x * lax.rsqrt(var + EPS)) * ln_ref[...]
    qkv = jnp.dot(xn, w_ref[...], preferred_element_type=jnp.float32)
    cos = cos_ref[...]
    sin = sin_ref[...]

    def rope_head(b):
        x1 = qkv[:, b:b + HALF]
        x2 = qkv[:, b + HALF:b + HD]
        return jnp.concatenate([x1 * cos - x2 * sin, x1 * sin + x2 * cos],
                               axis=1)

    for h in range(NH):
        q_ref[h] = rope_head(h * HD)
    for h in range(NKV):
        k_ref[h] = rope_head((NH + h) * HD)
        v_ref[h] = qkv[:, (NH + NKV + h) * HD:(NH + NKV + h + 1) * HD]


def _qkv_call(x, ln1_w, wqkv, cos, sin):
    return pl.pallas_call(
        _qkv_body,
        grid=(NR,),
        in_specs=[
            pl.BlockSpec((BR, H), lambda i: (i, 0)),
            pl.BlockSpec((1, H), lambda i: (0, 0)),
            pl.BlockSpec((H, (NH + 2 * NKV) * HD), lambda i: (0, 0)),
            pl.BlockSpec((BR, HALF), lambda i: (i, 0)),
            pl.BlockSpec((BR, HALF), lambda i: (i, 0)),
        ],
        out_specs=[
            pl.BlockSpec((NH, BR, HD), lambda i: (0, i, 0)),
            pl.BlockSpec((NKV, BR, HD), lambda i: (0, i, 0)),
            pl.BlockSpec((NKV, BR, HD), lambda i: (0, i, 0)),
        ],
        out_shape=[
            jax.ShapeDtypeStruct((NH, S, HD), jnp.float32),
            jax.ShapeDtypeStruct((NKV, S, HD), jnp.float32),
            jax.ShapeDtypeStruct((NKV, S, HD), jnp.float32),
        ],
    )(x, ln1_w, wqkv, cos, sin)


def _attn_body(sink_ref, q_ref, k_ref, v_ref, o_ref, m_ref, l_ref, acc_ref):
    h = pl.program_id(0)
    i = pl.program_id(1)
    j = pl.program_id(2)

    @pl.when(j == 0)
    def _init():
        m_ref[...] = jnp.full((BQ, 1), -1e30, jnp.float32)
        l_ref[...] = jnp.zeros((BQ, 1), jnp.float32)
        acc_ref[...] = jnp.zeros((BQ, HD), jnp.float32)

    @pl.when(j * BK < (i + 1) * BQ)
    def _compute():
        q = q_ref[0]
        k = k_ref[0]
        s = lax.dot_general(q, k, (((1,), (1,)), ((), ())),
                            preferred_element_type=jnp.float32)
        s = s * (HD ** -0.5)
        rows = i * BQ + lax.broadcasted_iota(jnp.int32, (BQ, BK), 0)
        cols = j * BK + lax.broadcasted_iota(jnp.int32, (BQ, BK), 1)
        s = jnp.where(rows >= cols, s, -1e30)
        m_prev = m_ref[...]
        m_cur = jnp.maximum(m_prev, jnp.max(s, axis=1, keepdims=True))
        alpha = jnp.exp(m_prev - m_cur)
        p = jnp.exp(s - m_cur)
        l_ref[...] = l_ref[...] * alpha + jnp.sum(p, axis=1, keepdims=True)
        acc_ref[...] = acc_ref[...] * alpha + jnp.dot(
            p, v_ref[0], preferred_element_type=jnp.float32)
        m_ref[...] = m_cur

    @pl.when(j == NJ - 1)
    def _fin():
        sink = sink_ref[h]
        l = l_ref[...] + jnp.exp(sink - m_ref[...])
        o_ref[0] = acc_ref[...] / l


def _attn_call(q, k, v, sinks):
    return pl.pallas_call(
        _attn_body,
        grid=(NH, NQ, NJ),
        in_specs=[
            pl.BlockSpec(memory_space=pltpu.SMEM),
            pl.BlockSpec((1, BQ, HD), lambda h, i, j: (h, i, 0)),
            pl.BlockSpec((1, BK, HD),
                         lambda h, i, j: (h // 2,
                                          jnp.minimum(j, (i * BQ + BQ - 1) // BK),
                                          0)),
            pl.BlockSpec((1, BK, HD),
                         lambda h, i, j: (h // 2,
                                          jnp.minimum(j, (i * BQ + BQ - 1) // BK),
                                          0)),
        ],
        out_specs=pl.BlockSpec((1, BQ, HD), lambda h, i, j: (h, i, 0)),
        out_shape=jax.ShapeDtypeStruct((NH, S, HD), jnp.float32),
        scratch_shapes=[
            pltpu.VMEM((BQ, 1), jnp.float32),
            pltpu.VMEM((BQ, 1), jnp.float32),
            pltpu.VMEM((BQ, HD), jnp.float32),
        ],
    )(sinks, q, k, v)


def _post_body(a_ref, wo_ref, hs_ref, ln_ref, rw_ref, rb_ref,
               hid_ref, h2_ref, oh1_ref, oh2_ref, w_ref):
    att = jnp.dot(a_ref[0], wo_ref[0], preferred_element_type=jnp.float32)
    for h in range(1, NH):
        att = att + jnp.dot(a_ref[h], wo_ref[h],
                            preferred_element_type=jnp.float32)
    hid = att + hs_ref[...]
    hid_ref[...] = hid
    var = jnp.mean(hid * hid, axis=-1, keepdims=True)
    h2 = (hid * lax.rsqrt(var + EPS)) * ln_ref[...]
    h2_ref[...] = h2
    lg = jnp.dot(h2, rw_ref[...],
                 preferred_element_type=jnp.float32) + rb_ref[...]
    iota_e = lax.broadcasted_iota(jnp.int32, (BR, E), 1)
    m1 = jnp.max(lg, axis=1, keepdims=True)
    i1 = jnp.min(jnp.where(lg >= m1, iota_e, E), axis=1, keepdims=True)
    oh1 = iota_e == i1
    lg2 = jnp.where(oh1, -1e30, lg)
    m2 = jnp.max(lg2, axis=1, keepdims=True)
    i2 = jnp.min(jnp.where(lg2 >= m2, iota_e, E), axis=1, keepdims=True)
    oh2 = iota_e == i2
    oh1_ref[...] = oh1.astype(jnp.int32)
    oh2_ref[...] = oh2.astype(jnp.int32)
    w1 = jax.nn.sigmoid(m1 - m2)
    w_ref[...] = jnp.concatenate([w1, 1.0 - w1], axis=1)


def _post_call(attn, wo, hs, ln2_w, router_w, router_b):
    return pl.pallas_call(
        _post_body,
        grid=(NR,),
        in_specs=[
            pl.BlockSpec((NH, BR, HD), lambda i: (0, i, 0)),
            pl.BlockSpec((NH, HD, H), lambda i: (0, 0, 0)),
            pl.BlockSpec((BR, H), lambda i: (i, 0)),
            pl.BlockSpec((1, H), lambda i: (0, 0)),
            pl.BlockSpec((H, E), lambda i: (0, 0)),
            pl.BlockSpec((1, E), lambda i: (0, 0)),
        ],
        out_specs=[
            pl.BlockSpec((BR, H), lambda i: (i, 0)),
            pl.BlockSpec((BR, H), lambda i: (i, 0)),
            pl.BlockSpec((BR, E), lambda i: (i, 0)),
            pl.BlockSpec((BR, E), lambda i: (i, 0)),
            pl.BlockSpec((BR, K), lambda i: (i, 0)),
        ],
        out_shape=[
            jax.ShapeDtypeStruct((S, H), jnp.float32),
            jax.ShapeDtypeStruct((S, H), jnp.float32),
            jax.ShapeDtypeStruct((S, E), jnp.int32),
            jax.ShapeDtypeStruct((S, E), jnp.int32),
            jax.ShapeDtypeStruct((S, K), jnp.float32),
        ],
    )(attn, wo, hs, ln2_w, router_w, router_b)


@functools.lru_cache(maxsize=None)
def _make_sc_gather(v_rows, d, b_rows, chunk):
    info = plsc.get_sparse_core_info()
    nc, ns = info.num_cores, info.num_subcores
    nw = nc * ns
    b_per_w = b_rows // nw
    nch = b_per_w // chunk
    mesh = plsc.VectorSubcoreMesh(core_axis_name="c", subcore_axis_name="s")

    @functools.partial(
        pl.kernel,
        mesh=mesh,
        out_type=jax.ShapeDtypeStruct((b_rows, d), jnp.float32),
        scratch_types=[
            pltpu.VMEM((chunk,), jnp.int32),
            pltpu.VMEM((chunk, d), jnp.float32),
            pltpu.SemaphoreType.DMA,
        ],
    )
    def gather(table_hbm, idx_hbm, out_hbm, idx_v, rows_v, sem):
        wid = lax.axis_index("s") * nc + lax.axis_index("c")
        base = wid * b_per_w

        def body(c, carry):
            off = base + c * chunk
            pltpu.sync_copy(idx_hbm.at[pl.ds(off, chunk)], idx_v)
            pltpu.async_copy(table_hbm.at[idx_v], rows_v, sem).wait()
            pltpu.sync_copy(rows_v, out_hbm.at[pl.ds(off, chunk)])
            return carry

        lax.fori_loop(0, nch, body, 0)

    return gather


@functools.lru_cache(maxsize=None)
def _make_sc_dispatch(d, n_slots, n_out, chunk):
    info = plsc.get_sparse_core_info()
    nc, ns = info.num_cores, info.num_subcores
    nw = nc * ns
    spw = n_slots // nw
    nch = spw // chunk
    mesh = plsc.VectorSubcoreMesh(core_axis_name="c", subcore_axis_name="s")

    @functools.partial(
        pl.kernel,
        mesh=mesh,
        out_type=jax.ShapeDtypeStruct((n_out, d), jnp.float32),
        scratch_types=[
            pltpu.VMEM((chunk,), jnp.int32),
            pltpu.VMEM((chunk,), jnp.int32),
            pltpu.VMEM((chunk, d), jnp.float32),
            pltpu.SemaphoreType.DMA,
        ],
    )
    def dispatch(table_hbm, sidx_hbm, didx_hbm, out_hbm,
                 sidx_v, didx_v, rows_v, sem):
        wid = lax.axis_index("s") * nc + lax.axis_index("c")
        base = wid * spw

        def body(c, carry):
            off = base + c * chunk
            pltpu.sync_copy(sidx_hbm.at[pl.ds(off, chunk)], sidx_v)
            pltpu.sync_copy(didx_hbm.at[pl.ds(off, chunk)], didx_v)
            pltpu.async_copy(table_hbm.at[sidx_v], rows_v, sem).wait()
            pltpu.async_copy(rows_v, out_hbm.at[didx_v], sem).wait()
            return carry

        lax.fori_loop(0, nch, body, 0)

    return dispatch


def _gmm_body(eid_ref, act_ref, xs_ref, guw_ref, gub_ref,
              dw_ref, db_ref, out_ref, dz_ref):
    t = pl.program_id(0)

    @pl.when(act_ref[t] == 1)
    def _():
        changed = (t == 0) | (eid_ref[t] != eid_ref[jnp.maximum(t - 1, 0)])

        @pl.when(changed)
        def _rebuild():
            dz_ref[...] = jnp.repeat(dw_ref[0], 2, axis=0)

        x = xs_ref[...]
        gu = jnp.dot(x, guw_ref[0],
                     preferred_element_type=jnp.float32) + gub_ref[0]
        g = jnp.minimum(gu, LIMIT)
        glu = g * jax.nn.sigmoid(g * ALPHA)
        u1 = jnp.clip(gu, -LIMIT, LIMIT) + 1.0
        lane = lax.broadcasted_iota(jnp.int32, (TM, 2 * F), 1)
        hh = jnp.where(lane % 2 == 0, glu * pltpu.roll(u1, 2 * F - 1, 1),
                       0.0)
        out_ref[...] = jnp.dot(hh, dz_ref[...],
                               preferred_element_type=jnp.float32) + db_ref[0]


def _gmm_call(tile_eid, tile_act, xs_pad, gate_up_w, gate_up_b,
              down_w, down_b):
    grid_spec = pltpu.PrefetchScalarGridSpec(
        num_scalar_prefetch=2,
        grid=(NT,),
        in_specs=[
            pl.BlockSpec((TM, H), lambda t, eid, act: (t, 0)),
            pl.BlockSpec((1, H, 2 * F), lambda t, eid, act: (eid[t], 0, 0)),
            pl.BlockSpec((1, 1, 2 * F), lambda t, eid, act: (eid[t], 0, 0)),
            pl.BlockSpec((1, F, H), lambda t, eid, act: (eid[t], 0, 0)),
            pl.BlockSpec((1, 1, H), lambda t, eid, act: (eid[t], 0, 0)),
        ],
        out_specs=pl.BlockSpec((TM, H), lambda t, eid, act: (t, 0)),
        scratch_shapes=[pltpu.VMEM((2 * F, H), jnp.float32)],
    )
    return pl.pallas_call(
        _gmm_body,
        grid_spec=grid_spec,
        out_shape=jax.ShapeDtypeStruct((NPAD, H), jnp.float32),
    )(tile_eid, tile_act, xs_pad, gate_up_w, gate_up_b, down_w, down_b)


def _comb_body(hid_ref, sr_ref, w_ref, out_ref):
    srv = sr_ref[...]
    w = w_ref[...]
    out_ref[...] = (hid_ref[...] + w[:, 0:1] * srv[:, :H]
                    + w[:, 1:2] * srv[:, H:])


def _comb_call(hidden, slot_rows, w):
    return pl.pallas_call(
        _comb_body,
        grid=(NR,),
        in_specs=[
            pl.BlockSpec((BR, H), lambda i: (i, 0)),
            pl.BlockSpec((BR, 2 * H), lambda i: (i, 0)),
            pl.BlockSpec((BR, K), lambda i: (i, 0)),
        ],
        out_specs=pl.BlockSpec((BR, H), lambda i: (i, 0)),
        out_shape=jax.ShapeDtypeStruct((S, H), jnp.float32),
    )(hidden, slot_rows, w)


def _routing(oh1, oh2):
    oh = jnp.stack([oh1, oh2], axis=1).reshape(S * K, E)
    incl = jnp.cumsum(oh, axis=0)
    counts = incl[-1]
    rank = jnp.sum((incl - oh) * oh, axis=1)
    padded = ((counts + TM - 1) // TM) * TM
    ends_pad = jnp.cumsum(padded)
    padoff = ends_pad - padded
    pos_flat = (jnp.sum(oh * padoff[None, :], axis=1) + rank).astype(jnp.int32)
    tstart = jnp.arange(NT, dtype=jnp.int32)[:, None] * TM
    tile_eid = jnp.sum((tstart >= ends_pad[None, :]).astype(jnp.int32), axis=1)
    tile_eid_c = jnp.minimum(tile_eid, E - 1)
    toh = (tile_eid_c[:, None] == jnp.arange(E, dtype=jnp.int32)[None, :]
           ).astype(jnp.int32)
    tile_act = (((tstart[:, 0] - jnp.sum(toh * padoff[None, :], axis=1))
                 < jnp.sum(toh * counts[None, :], axis=1))
                & (tile_eid < E)).astype(jnp.int32)
    return pos_flat, tile_eid_c, tile_act


def kernel(hidden_states, positions, ln1_w, wq, wk, wv, wo, sinks, ln2_w,
           router_w, router_b, gate_up_w, gate_up_b, down_w, down_b):
    x = hidden_states
    wqkv = jnp.concatenate([wq, wk, wv], axis=1)
    inv = 1.0 / (THETA ** (jnp.arange(HALF, dtype=jnp.float32) / HALF))
    ang = positions.astype(jnp.float32)[:, None] * inv[None, :]
    cos = jnp.cos(ang)
    sin = jnp.sin(ang)

    q, k, v = _qkv_call(x, ln1_w.reshape(1, H), wqkv, cos, sin)
    attn = _attn_call(q, k, v, sinks)
    hidden, h2, oh1, oh2, w = _post_call(attn, wo.reshape(NH, HD, H), x,
                                         ln2_w.reshape(1, H),
                                         router_w, router_b.reshape(1, E))

    pos_flat, tile_eid, tile_act = _routing(oh1, oh2)

    src_slot = (jnp.arange(S * K, dtype=jnp.int32) // K)
    xs_pad = _make_sc_dispatch(H, S * K, NPAD, 32)(h2, src_slot, pos_flat)
    rows = _gmm_call(tile_eid, tile_act, xs_pad, gate_up_w,
                     gate_up_b.reshape(E, 1, 2 * F), down_w,
                     down_b.reshape(E, 1, H))
    slot_rows = _make_sc_gather(NPAD, H, S * K, 32)(rows, pos_flat)
    return _comb_call(hidden, slot_rows.reshape(S, 2 * H), w)


# speedup vs baseline: 1.5038x; 1.0469x over previous
import functools

import jax
import jax.numpy as jnp
from jax import lax
from jax.experimental import pallas as pl
from jax.experimental.pallas import tpu as pltpu
from jax.experimental.pallas import tpu_sc as plsc

S = 2048
H = 1024
NH = 16
NKV = 8
HD = 64
E = 8
K = 2
F = 1024
EPS = 1e-5
THETA = 150000.0
ALPHA = 1.702
LIMIT = 7.0
HALF = HD // 2

BQ = 1024
BK = 1024
NQ = S // BQ
NJ = S // BK
BR = 256
NR = S // BR
TM = 128
NT = (S * K) // TM + E
NPAD = NT * TM


def _qkv_body(x_ref, ln_ref, w_ref, cos_ref, sin_ref, q_ref, k_ref, v_ref):
    x = x_ref[...]
    var = jnp.mean(x * x, axis=-1, keepdims=True)
    xn = (x * lax.rsqrt(var + EPS)) * ln_ref[...]
    qkv = jnp.dot(xn, w_ref[...], preferred_element_type=jnp.float32)
    cos = cos_ref[...]
    sin = sin_ref[...]

    def rope_head(b):
        x1 = qkv[:, b:b + HALF]
        x2 = qkv[:, b + HALF:b + HD]
        return jnp.concatenate([x1 * cos - x2 * sin, x1 * sin + x2 * cos],
                               axis=1)

    for h in range(NH):
        q_ref[h] = rope_head(h * HD)
    for h in range(NKV):
        k_ref[h] = rope_head((NH + h) * HD)
        v_ref[h] = qkv[:, (NH + NKV + h) * HD:(NH + NKV + h + 1) * HD]


def _qkv_call(x, ln1_w, wqkv, cos, sin):
    return pl.pallas_call(
        _qkv_body,
        grid=(NR,),
        in_specs=[
            pl.BlockSpec((BR, H), lambda i: (i, 0)),
            pl.BlockSpec((1, H), lambda i: (0, 0)),
            pl.BlockSpec((H, (NH + 2 * NKV) * HD), lambda i: (0, 0)),
            pl.BlockSpec((BR, HALF), lambda i: (i, 0)),
            pl.BlockSpec((BR, HALF), lambda i: (i, 0)),
        ],
        out_specs=[
            pl.BlockSpec((NH, BR, HD), lambda i: (0, i, 0)),
            pl.BlockSpec((NKV, BR, HD), lambda i: (0, i, 0)),
            pl.BlockSpec((NKV, BR, HD), lambda i: (0, i, 0)),
        ],
        out_shape=[
            jax.ShapeDtypeStruct((NH, S, HD), jnp.float32),
            jax.ShapeDtypeStruct((NKV, S, HD), jnp.float32),
            jax.ShapeDtypeStruct((NKV, S, HD), jnp.float32),
        ],
    )(x, ln1_w, wqkv, cos, sin)


def _attn_body(sink_ref, q_ref, k_ref, v_ref, o_ref, m_ref, l_ref, acc_ref):
    h = pl.program_id(0)
    i = pl.program_id(1)
    j = pl.program_id(2)

    @pl.when(j == 0)
    def _init():
        m_ref[...] = jnp.full((BQ, 1), -1e30, jnp.float32)
        l_ref[...] = jnp.zeros((BQ, 1), jnp.float32)
        acc_ref[...] = jnp.zeros((BQ, HD), jnp.float32)

    @pl.when(j * BK < (i + 1) * BQ)
    def _compute():
        q = q_ref[0]
        k = k_ref[0]
        s = lax.dot_general(q, k, (((1,), (1,)), ((), ())),
                            preferred_element_type=jnp.float32)
        s = s * (HD ** -0.5)
        rows = i * BQ + lax.broadcasted_iota(jnp.int32, (BQ, BK), 0)
        cols = j * BK + lax.broadcasted_iota(jnp.int32, (BQ, BK), 1)
        s = jnp.where(rows >= cols, s, -1e30)
        m_prev = m_ref[...]
        m_cur = jnp.maximum(m_prev, jnp.max(s, axis=1, keepdims=True))
        alpha = jnp.exp(m_prev - m_cur)
        p = jnp.exp(s - m_cur)
        l_ref[...] = l_ref[...] * alpha + jnp.sum(p, axis=1, keepdims=True)
        acc_ref[...] = acc_ref[...] * alpha + jnp.dot(
            p, v_ref[0], preferred_element_type=jnp.float32)
        m_ref[...] = m_cur

    @pl.when(j == NJ - 1)
    def _fin():
        sink = sink_ref[h]
        l = l_ref[...] + jnp.exp(sink - m_ref[...])
        o_ref[0] = acc_ref[...] / l


def _attn_call(q, k, v, sinks):
    return pl.pallas_call(
        _attn_body,
        grid=(NH, NQ, NJ),
        in_specs=[
            pl.BlockSpec(memory_space=pltpu.SMEM),
            pl.BlockSpec((1, BQ, HD), lambda h, i, j: (h, i, 0)),
            pl.BlockSpec((1, BK, HD),
                         lambda h, i, j: (h // 2,
                                          jnp.minimum(j, (i * BQ + BQ - 1) // BK),
                                          0)),
            pl.BlockSpec((1, BK, HD),
                         lambda h, i, j: (h // 2,
                                          jnp.minimum(j, (i * BQ + BQ - 1) // BK),
                                          0)),
        ],
        out_specs=pl.BlockSpec((1, BQ, HD), lambda h, i, j: (h, i, 0)),
        out_shape=jax.ShapeDtypeStruct((NH, S, HD), jnp.float32),
        scratch_shapes=[
            pltpu.VMEM((BQ, 1), jnp.float32),
            pltpu.VMEM((BQ, 1), jnp.float32),
            pltpu.VMEM((BQ, HD), jnp.float32),
        ],
    )(sinks, q, k, v)


def _post_body(a_ref, wo_ref, hs_ref, ln_ref, rw_ref, rb_ref,
               hid_ref, h2_ref, oh1_ref, oh2_ref, w_ref):
    att = jnp.dot(a_ref[0], wo_ref[0], preferred_element_type=jnp.float32)
    for h in range(1, NH):
        att = att + jnp.dot(a_ref[h], wo_ref[h],
                            preferred_element_type=jnp.float32)
    hid = att + hs_ref[...]
    hid_ref[...] = hid
    var = jnp.mean(hid * hid, axis=-1, keepdims=True)
    h2 = (hid * lax.rsqrt(var + EPS)) * ln_ref[...]
    h2_ref[...] = h2
    lg = jnp.dot(h2, rw_ref[...],
                 preferred_element_type=jnp.float32) + rb_ref[...]
    iota_e = lax.broadcasted_iota(jnp.int32, (BR, E), 1)
    m1 = jnp.max(lg, axis=1, keepdims=True)
    i1 = jnp.min(jnp.where(lg >= m1, iota_e, E), axis=1, keepdims=True)
    oh1 = iota_e == i1
    lg2 = jnp.where(oh1, -1e30, lg)
    m2 = jnp.max(lg2, axis=1, keepdims=True)
    i2 = jnp.min(jnp.where(lg2 >= m2, iota_e, E), axis=1, keepdims=True)
    oh2 = iota_e == i2
    oh1_ref[...] = oh1.astype(jnp.int32)
    oh2_ref[...] = oh2.astype(jnp.int32)
    w1 = jax.nn.sigmoid(m1 - m2)
    w_ref[...] = jnp.concatenate([w1, 1.0 - w1], axis=1)


def _post_call(attn, wo, hs, ln2_w, router_w, router_b):
    return pl.pallas_call(
        _post_body,
        grid=(NR,),
        in_specs=[
            pl.BlockSpec((NH, BR, HD), lambda i: (0, i, 0)),
            pl.BlockSpec((NH, HD, H), lambda i: (0, 0, 0)),
            pl.BlockSpec((BR, H), lambda i: (i, 0)),
            pl.BlockSpec((1, H), lambda i: (0, 0)),
            pl.BlockSpec((H, E), lambda i: (0, 0)),
            pl.BlockSpec((1, E), lambda i: (0, 0)),
        ],
        out_specs=[
            pl.BlockSpec((BR, H), lambda i: (i, 0)),
            pl.BlockSpec((BR, H), lambda i: (i, 0)),
            pl.BlockSpec((BR, E), lambda i: (i, 0)),
            pl.BlockSpec((BR, E), lambda i: (i, 0)),
            pl.BlockSpec((BR, K), lambda i: (i, 0)),
        ],
        out_shape=[
            jax.ShapeDtypeStruct((S, H), jnp.float32),
            jax.ShapeDtypeStruct((S, H), jnp.float32),
            jax.ShapeDtypeStruct((S, E), jnp.int32),
            jax.ShapeDtypeStruct((S, E), jnp.int32),
            jax.ShapeDtypeStruct((S, K), jnp.float32),
        ],
    )(attn, wo, hs, ln2_w, router_w, router_b)


@functools.lru_cache(maxsize=None)
def _make_sc_return(d, n_rows, chunk):
    info = plsc.get_sparse_core_info()
    nc, ns = info.num_cores, info.num_subcores
    nw = nc * ns
    b_per_w = n_rows // nw
    nch = b_per_w // chunk
    mesh = plsc.VectorSubcoreMesh(core_axis_name="c", subcore_axis_name="s")

    @functools.partial(
        pl.kernel,
        mesh=mesh,
        out_type=[
            jax.ShapeDtypeStruct((n_rows, d), jnp.float32),
            jax.ShapeDtypeStruct((n_rows, d), jnp.float32),
        ],
        scratch_types=[
            pltpu.VMEM((chunk,), jnp.int32),
            pltpu.VMEM((chunk,), jnp.int32),
            pltpu.VMEM((chunk, d), jnp.float32),
            pltpu.VMEM((chunk, d), jnp.float32),
            pltpu.SemaphoreType.DMA,
            pltpu.SemaphoreType.DMA,
        ],
    )
    def gather(table_hbm, idx1_hbm, idx2_hbm, out1_hbm, out2_hbm,
               i1_v, i2_v, r1_v, r2_v, sem1, sem2):
        wid = lax.axis_index("s") * nc + lax.axis_index("c")
        base = wid * b_per_w

        def body(c, carry):
            off = base + c * chunk
            pltpu.sync_copy(idx1_hbm.at[pl.ds(off, chunk)], i1_v)
            pltpu.sync_copy(idx2_hbm.at[pl.ds(off, chunk)], i2_v)
            c1 = pltpu.async_copy(table_hbm.at[i1_v], r1_v, sem1)
            c2 = pltpu.async_copy(table_hbm.at[i2_v], r2_v, sem2)
            c1.wait()
            c2.wait()
            pltpu.sync_copy(r1_v, out1_hbm.at[pl.ds(off, chunk)])
            pltpu.sync_copy(r2_v, out2_hbm.at[pl.ds(off, chunk)])
            return carry

        lax.fori_loop(0, nch, body, 0)

    return gather


@functools.lru_cache(maxsize=None)
def _make_sc_dispatch(d, n_slots, n_out, chunk):
    info = plsc.get_sparse_core_info()
    nc, ns = info.num_cores, info.num_subcores
    nw = nc * ns
    spw = n_slots // nw
    nch = spw // chunk
    mesh = plsc.VectorSubcoreMesh(core_axis_name="c", subcore_axis_name="s")

    @functools.partial(
        pl.kernel,
        mesh=mesh,
        out_type=jax.ShapeDtypeStruct((n_out, d), jnp.float32),
        scratch_types=[
            pltpu.VMEM((chunk,), jnp.int32),
            pltpu.VMEM((chunk,), jnp.int32),
            pltpu.VMEM((chunk, d), jnp.float32),
            pltpu.SemaphoreType.DMA,
        ],
    )
    def dispatch(table_hbm, sidx_hbm, didx_hbm, out_hbm,
                 sidx_v, didx_v, rows_v, sem):
        wid = lax.axis_index("s") * nc + lax.axis_index("c")
        base = wid * spw

        def body(c, carry):
            off = base + c * chunk
            pltpu.sync_copy(sidx_hbm.at[pl.ds(off, chunk)], sidx_v)
            pltpu.sync_copy(didx_hbm.at[pl.ds(off, chunk)], didx_v)
            pltpu.async_copy(table_hbm.at[sidx_v], rows_v, sem).wait()
            pltpu.async_copy(rows_v, out_hbm.at[didx_v], sem).wait()
            return carry

        lax.fori_loop(0, nch, body, 0)

    return dispatch


def _gmm_body(eid_ref, act_ref, xs_ref, guw_ref, gub_ref,
              dw_ref, db_ref, out_ref, dz_ref):
    t = pl.program_id(0)

    @pl.when(act_ref[t] == 1)
    def _():
        changed = (t == 0) | (eid_ref[t] != eid_ref[jnp.maximum(t - 1, 0)])

        @pl.when(changed)
        def _rebuild():
            dz_ref[...] = jnp.repeat(dw_ref[0], 2, axis=0)

        x = xs_ref[...]
        gu = jnp.dot(x, guw_ref[0],
                     preferred_element_type=jnp.float32) + gub_ref[0]
        g = jnp.minimum(gu, LIMIT)
        glu = g * jax.nn.sigmoid(g * ALPHA)
        u1 = jnp.clip(gu, -LIMIT, LIMIT) + 1.0
        lane = lax.broadcasted_iota(jnp.int32, (TM, 2 * F), 1)
        hh = jnp.where(lane % 2 == 0, glu * pltpu.roll(u1, 2 * F - 1, 1),
                       0.0)
        out_ref[...] = jnp.dot(hh, dz_ref[...],
                               preferred_element_type=jnp.float32) + db_ref[0]


def _gmm_call(tile_eid, tile_act, xs_pad, gate_up_w, gate_up_b,
              down_w, down_b):
    grid_spec = pltpu.PrefetchScalarGridSpec(
        num_scalar_prefetch=2,
        grid=(NT,),
        in_specs=[
            pl.BlockSpec((TM, H), lambda t, eid, act: (t, 0)),
            pl.BlockSpec((1, H, 2 * F), lambda t, eid, act: (eid[t], 0, 0)),
            pl.BlockSpec((1, 1, 2 * F), lambda t, eid, act: (eid[t], 0, 0)),
            pl.BlockSpec((1, F, H), lambda t, eid, act: (eid[t], 0, 0)),
            pl.BlockSpec((1, 1, H), lambda t, eid, act: (eid[t], 0, 0)),
        ],
        out_specs=pl.BlockSpec((TM, H), lambda t, eid, act: (t, 0)),
        scratch_shapes=[pltpu.VMEM((2 * F, H), jnp.float32)],
    )
    return pl.pallas_call(
        _gmm_body,
        grid_spec=grid_spec,
        out_shape=jax.ShapeDtypeStruct((NPAD, H), jnp.float32),
    )(tile_eid, tile_act, xs_pad, gate_up_w, gate_up_b, down_w, down_b)


def _comb_body(hid_ref, t1_ref, t2_ref, w_ref, out_ref):
    w = w_ref[...]
    out_ref[...] = (hid_ref[...] + w[:, 0:1] * t1_ref[...]
                    + w[:, 1:2] * t2_ref[...])


def _comb_call(hidden, top1, top2, w):
    return pl.pallas_call(
        _comb_body,
        grid=(NR,),
        in_specs=[
            pl.BlockSpec((BR, H), lambda i: (i, 0)),
            pl.BlockSpec((BR, H), lambda i: (i, 0)),
            pl.BlockSpec((BR, H), lambda i: (i, 0)),
            pl.BlockSpec((BR, K), lambda i: (i, 0)),
        ],
        out_specs=pl.BlockSpec((BR, H), lambda i: (i, 0)),
        out_shape=jax.ShapeDtypeStruct((S, H), jnp.float32),
    )(hidden, top1, top2, w)


def _routing(oh1, oh2):
    oh = jnp.stack([oh1, oh2], axis=1).reshape(S * K, E)
    incl = jnp.cumsum(oh, axis=0)
    counts = incl[-1]
    rank = jnp.sum((incl - oh) * oh, axis=1)
    padded = ((counts + TM - 1) // TM) * TM
    ends_pad = jnp.cumsum(padded)
    padoff = ends_pad - padded
    pos_flat = (jnp.sum(oh * padoff[None, :], axis=1) + rank).astype(jnp.int32)
    tstart = jnp.arange(NT, dtype=jnp.int32)[:, None] * TM
    tile_eid = jnp.sum((tstart >= ends_pad[None, :]).astype(jnp.int32), axis=1)
    tile_eid_c = jnp.minimum(tile_eid, E - 1)
    toh = (tile_eid_c[:, None] == jnp.arange(E, dtype=jnp.int32)[None, :]
           ).astype(jnp.int32)
    tile_act = (((tstart[:, 0] - jnp.sum(toh * padoff[None, :], axis=1))
                 < jnp.sum(toh * counts[None, :], axis=1))
                & (tile_eid < E)).astype(jnp.int32)
    return pos_flat, tile_eid_c, tile_act


def kernel(hidden_states, positions, ln1_w, wq, wk, wv, wo, sinks, ln2_w,
           router_w, router_b, gate_up_w, gate_up_b, down_w, down_b):
    x = hidden_states
    wqkv = jnp.concatenate([wq, wk, wv], axis=1)
    inv = 1.0 / (THETA ** (jnp.arange(HALF, dtype=jnp.float32) / HALF))
    ang = positions.astype(jnp.float32)[:, None] * inv[None, :]
    cos = jnp.cos(ang)
    sin = jnp.sin(ang)

    q, k, v = _qkv_call(x, ln1_w.reshape(1, H), wqkv, cos, sin)
    attn = _attn_call(q, k, v, sinks)
    hidden, h2, oh1, oh2, w = _post_call(attn, wo.reshape(NH, HD, H), x,
                                         ln2_w.reshape(1, H),
                                         router_w, router_b.reshape(1, E))

    pos_flat, tile_eid, tile_act = _routing(oh1, oh2)

    src_slot = (jnp.arange(S * K, dtype=jnp.int32) // K)
    xs_pad = _make_sc_dispatch(H, S * K, NPAD, 32)(h2, src_slot, pos_flat)
    rows = _gmm_call(tile_eid, tile_act, xs_pad, gate_up_w,
                     gate_up_b.reshape(E, 1, 2 * F), down_w,
                     down_b.reshape(E, 1, H))
    pos2 = pos_flat.reshape(S, K)
    top1, top2 = _make_sc_return(H, S, 32)(rows, pos2[:, 0], pos2[:, 1])
    return _comb_call(hidden, top1, top2, w)


# speedup vs baseline: 1.5423x; 1.0256x over previous
import functools

import jax
import jax.numpy as jnp
from jax import lax
from jax.experimental import pallas as pl
from jax.experimental.pallas import tpu as pltpu
from jax.experimental.pallas import tpu_sc as plsc

S = 2048
H = 1024
NH = 16
NKV = 8
HD = 64
E = 8
K = 2
F = 1024
EPS = 1e-5
THETA = 150000.0
ALPHA = 1.702
LIMIT = 7.0
HALF = HD // 2

BQ = 1024
BK = 1024
NQ = S // BQ
NJ = S // BK
BR = 256
NR = S // BR
TM = 256
NT = (S * K) // TM + E
NPAD = NT * TM


def _qkv_body(x_ref, ln_ref, w_ref, cos_ref, sin_ref, q_ref, k_ref, v_ref):
    x = x_ref[...]
    var = jnp.mean(x * x, axis=-1, keepdims=True)
    xn = (x * lax.rsqrt(var + EPS)) * ln_ref[...]
    qkv = jnp.dot(xn, w_ref[...], preferred_element_type=jnp.float32)
    cos = cos_ref[...]
    sin = sin_ref[...]

    def rope_head(b):
        x1 = qkv[:, b:b + HALF]
        x2 = qkv[:, b + HALF:b + HD]
        return jnp.concatenate([x1 * cos - x2 * sin, x1 * sin + x2 * cos],
                               axis=1)

    for h in range(NH):
        q_ref[h] = rope_head(h * HD)
    for h in range(NKV):
        k_ref[h] = rope_head((NH + h) * HD)
        v_ref[h] = qkv[:, (NH + NKV + h) * HD:(NH + NKV + h + 1) * HD]


def _qkv_call(x, ln1_w, wqkv, cos, sin):
    return pl.pallas_call(
        _qkv_body,
        grid=(NR,),
        in_specs=[
            pl.BlockSpec((BR, H), lambda i: (i, 0)),
            pl.BlockSpec((1, H), lambda i: (0, 0)),
            pl.BlockSpec((H, (NH + 2 * NKV) * HD), lambda i: (0, 0)),
            pl.BlockSpec((BR, HALF), lambda i: (i, 0)),
            pl.BlockSpec((BR, HALF), lambda i: (i, 0)),
        ],
        out_specs=[
            pl.BlockSpec((NH, BR, HD), lambda i: (0, i, 0)),
            pl.BlockSpec((NKV, BR, HD), lambda i: (0, i, 0)),
            pl.BlockSpec((NKV, BR, HD), lambda i: (0, i, 0)),
        ],
        out_shape=[
            jax.ShapeDtypeStruct((NH, S, HD), jnp.float32),
            jax.ShapeDtypeStruct((NKV, S, HD), jnp.float32),
            jax.ShapeDtypeStruct((NKV, S, HD), jnp.float32),
        ],
    )(x, ln1_w, wqkv, cos, sin)


def _attn_body(sink_ref, q_ref, k_ref, v_ref, o_ref, m_ref, l_ref, acc_ref):
    h = pl.program_id(0)
    i = pl.program_id(1)
    j = pl.program_id(2)

    @pl.when(j == 0)
    def _init():
        m_ref[...] = jnp.full((BQ, 1), -1e30, jnp.float32)
        l_ref[...] = jnp.zeros((BQ, 1), jnp.float32)
        acc_ref[...] = jnp.zeros((BQ, HD), jnp.float32)

    @pl.when(j * BK < (i + 1) * BQ)
    def _compute():
        q = q_ref[0]
        k = k_ref[0]
        s = lax.dot_general(q, k, (((1,), (1,)), ((), ())),
                            preferred_element_type=jnp.float32)
        s = s * (HD ** -0.5)
        rows = i * BQ + lax.broadcasted_iota(jnp.int32, (BQ, BK), 0)
        cols = j * BK + lax.broadcasted_iota(jnp.int32, (BQ, BK), 1)
        s = jnp.where(rows >= cols, s, -1e30)
        m_prev = m_ref[...]
        m_cur = jnp.maximum(m_prev, jnp.max(s, axis=1, keepdims=True))
        alpha = jnp.exp(m_prev - m_cur)
        p = jnp.exp(s - m_cur)
        l_ref[...] = l_ref[...] * alpha + jnp.sum(p, axis=1, keepdims=True)
        acc_ref[...] = acc_ref[...] * alpha + jnp.dot(
            p, v_ref[0], preferred_element_type=jnp.float32)
        m_ref[...] = m_cur

    @pl.when(j == NJ - 1)
    def _fin():
        sink = sink_ref[h]
        l = l_ref[...] + jnp.exp(sink - m_ref[...])
        o_ref[0] = acc_ref[...] / l


def _attn_call(q, k, v, sinks):
    return pl.pallas_call(
        _attn_body,
        grid=(NH, NQ, NJ),
        in_specs=[
            pl.BlockSpec(memory_space=pltpu.SMEM),
            pl.BlockSpec((1, BQ, HD), lambda h, i, j: (h, i, 0)),
            pl.BlockSpec((1, BK, HD),
                         lambda h, i, j: (h // 2,
                                          jnp.minimum(j, (i * BQ + BQ - 1) // BK),
                                          0)),
            pl.BlockSpec((1, BK, HD),
                         lambda h, i, j: (h // 2,
                                          jnp.minimum(j, (i * BQ + BQ - 1) // BK),
                                          0)),
        ],
        out_specs=pl.BlockSpec((1, BQ, HD), lambda h, i, j: (h, i, 0)),
        out_shape=jax.ShapeDtypeStruct((NH, S, HD), jnp.float32),
        scratch_shapes=[
            pltpu.VMEM((BQ, 1), jnp.float32),
            pltpu.VMEM((BQ, 1), jnp.float32),
            pltpu.VMEM((BQ, HD), jnp.float32),
        ],
    )(sinks, q, k, v)


def _post_body(a_ref, wo_ref, hs_ref, ln_ref, rw_ref, rb_ref,
               hid_ref, h2_ref, oh1_ref, oh2_ref, w_ref):
    att = jnp.dot(a_ref[0], wo_ref[0], preferred_element_type=jnp.float32)
    for h in range(1, NH):
        att = att + jnp.dot(a_ref[h], wo_ref[h],
                            preferred_element_type=jnp.float32)
    hid = att + hs_ref[...]
    hid_ref[...] = hid
    var = jnp.mean(hid * hid, axis=-1, keepdims=True)
    h2 = (hid * lax.rsqrt(var + EPS)) * ln_ref[...]
    h2_ref[...] = h2
    lg = jnp.dot(h2, rw_ref[...],
                 preferred_element_type=jnp.float32) + rb_ref[...]
    iota_e = lax.broadcasted_iota(jnp.int32, (BR, E), 1)
    m1 = jnp.max(lg, axis=1, keepdims=True)
    i1 = jnp.min(jnp.where(lg >= m1, iota_e, E), axis=1, keepdims=True)
    oh1 = iota_e == i1
    lg2 = jnp.where(oh1, -1e30, lg)
    m2 = jnp.max(lg2, axis=1, keepdims=True)
    i2 = jnp.min(jnp.where(lg2 >= m2, iota_e, E), axis=1, keepdims=True)
    oh2 = iota_e == i2
    oh1_ref[...] = oh1.astype(jnp.int32)
    oh2_ref[...] = oh2.astype(jnp.int32)
    w1 = jax.nn.sigmoid(m1 - m2)
    w_ref[...] = jnp.concatenate([w1, 1.0 - w1], axis=1)


def _post_call(attn, wo, hs, ln2_w, router_w, router_b):
    return pl.pallas_call(
        _post_body,
        grid=(NR,),
        in_specs=[
            pl.BlockSpec((NH, BR, HD), lambda i: (0, i, 0)),
            pl.BlockSpec((NH, HD, H), lambda i: (0, 0, 0)),
            pl.BlockSpec((BR, H), lambda i: (i, 0)),
            pl.BlockSpec((1, H), lambda i: (0, 0)),
            pl.BlockSpec((H, E), lambda i: (0, 0)),
            pl.BlockSpec((1, E), lambda i: (0, 0)),
        ],
        out_specs=[
            pl.BlockSpec((BR, H), lambda i: (i, 0)),
            pl.BlockSpec((BR, H), lambda i: (i, 0)),
            pl.BlockSpec((BR, E), lambda i: (i, 0)),
            pl.BlockSpec((BR, E), lambda i: (i, 0)),
            pl.BlockSpec((BR, K), lambda i: (i, 0)),
        ],
        out_shape=[
            jax.ShapeDtypeStruct((S, H), jnp.float32),
            jax.ShapeDtypeStruct((S, H), jnp.float32),
            jax.ShapeDtypeStruct((S, E), jnp.int32),
            jax.ShapeDtypeStruct((S, E), jnp.int32),
            jax.ShapeDtypeStruct((S, K), jnp.float32),
        ],
    )(attn, wo, hs, ln2_w, router_w, router_b)


@functools.lru_cache(maxsize=None)
def _make_sc_return(d, n_rows, chunk):
    info = plsc.get_sparse_core_info()
    nc, ns = info.num_cores, info.num_subcores
    nw = nc * ns
    b_per_w = n_rows // nw
    nch = b_per_w // chunk
    mesh = plsc.VectorSubcoreMesh(core_axis_name="c", subcore_axis_name="s")

    @functools.partial(
        pl.kernel,
        mesh=mesh,
        out_type=[
            jax.ShapeDtypeStruct((n_rows, d), jnp.float32),
            jax.ShapeDtypeStruct((n_rows, d), jnp.float32),
        ],
        scratch_types=[
            pltpu.VMEM((chunk,), jnp.int32),
            pltpu.VMEM((chunk,), jnp.int32),
            pltpu.VMEM((chunk, d), jnp.float32),
            pltpu.VMEM((chunk, d), jnp.float32),
            pltpu.SemaphoreType.DMA,
            pltpu.SemaphoreType.DMA,
        ],
    )
    def gather(table_hbm, idx1_hbm, idx2_hbm, out1_hbm, out2_hbm,
               i1_v, i2_v, r1_v, r2_v, sem1, sem2):
        wid = lax.axis_index("s") * nc + lax.axis_index("c")
        base = wid * b_per_w

        def body(c, carry):
            off = base + c * chunk
            pltpu.sync_copy(idx1_hbm.at[pl.ds(off, chunk)], i1_v)
            pltpu.sync_copy(idx2_hbm.at[pl.ds(off, chunk)], i2_v)
            c1 = pltpu.async_copy(table_hbm.at[i1_v], r1_v, sem1)
            c2 = pltpu.async_copy(table_hbm.at[i2_v], r2_v, sem2)
            c1.wait()
            c2.wait()
            pltpu.sync_copy(r1_v, out1_hbm.at[pl.ds(off, chunk)])
            pltpu.sync_copy(r2_v, out2_hbm.at[pl.ds(off, chunk)])
            return carry

        lax.fori_loop(0, nch, body, 0)

    return gather


@functools.lru_cache(maxsize=None)
def _make_sc_dispatch(d, n_slots, n_out, chunk):
    info = plsc.get_sparse_core_info()
    nc, ns = info.num_cores, info.num_subcores
    nw = nc * ns
    spw = n_slots // nw
    nch = spw // chunk
    mesh = plsc.VectorSubcoreMesh(core_axis_name="c", subcore_axis_name="s")

    @functools.partial(
        pl.kernel,
        mesh=mesh,
        out_type=jax.ShapeDtypeStruct((n_out, d), jnp.float32),
        scratch_types=[
            pltpu.VMEM((chunk,), jnp.int32),
            pltpu.VMEM((chunk,), jnp.int32),
            pltpu.VMEM((chunk, d), jnp.float32),
            pltpu.SemaphoreType.DMA,
        ],
    )
    def dispatch(table_hbm, sidx_hbm, didx_hbm, out_hbm,
                 sidx_v, didx_v, rows_v, sem):
        wid = lax.axis_index("s") * nc + lax.axis_index("c")
        base = wid * spw

        def body(c, carry):
            off = base + c * chunk
            pltpu.sync_copy(sidx_hbm.at[pl.ds(off, chunk)], sidx_v)
            pltpu.sync_copy(didx_hbm.at[pl.ds(off, chunk)], didx_v)
            pltpu.async_copy(table_hbm.at[sidx_v], rows_v, sem).wait()
            pltpu.async_copy(rows_v, out_hbm.at[didx_v], sem).wait()
            return carry

        lax.fori_loop(0, nch, body, 0)

    return dispatch


def _gmm_body(eid_ref, act_ref, xs_ref, guw_ref, gub_ref,
              dw_ref, db_ref, out_ref, dz_ref):
    t = pl.program_id(0)

    @pl.when(act_ref[t] == 1)
    def _():
        changed = (t == 0) | (eid_ref[t] != eid_ref[jnp.maximum(t - 1, 0)])

        @pl.when(changed)
        def _rebuild():
            dz_ref[...] = jnp.repeat(dw_ref[0], 2, axis=0)

        x = xs_ref[...]
        gu = jnp.dot(x, guw_ref[0],
                     preferred_element_type=jnp.float32) + gub_ref[0]
        g = jnp.minimum(gu, LIMIT)
        glu = g * jax.nn.sigmoid(g * ALPHA)
        u1 = jnp.clip(gu, -LIMIT, LIMIT) + 1.0
        lane = lax.broadcasted_iota(jnp.int32, (TM, 2 * F), 1)
        hh = jnp.where(lane % 2 == 0, glu * pltpu.roll(u1, 2 * F - 1, 1),
                       0.0)
        out_ref[...] = jnp.dot(hh, dz_ref[...],
                               preferred_element_type=jnp.float32) + db_ref[0]


def _gmm_call(tile_eid, tile_act, xs_pad, gate_up_w, gate_up_b,
              down_w, down_b):
    grid_spec = pltpu.PrefetchScalarGridSpec(
        num_scalar_prefetch=2,
        grid=(NT,),
        in_specs=[
            pl.BlockSpec((TM, H), lambda t, eid, act: (t, 0)),
            pl.BlockSpec((1, H, 2 * F), lambda t, eid, act: (eid[t], 0, 0)),
            pl.BlockSpec((1, 1, 2 * F), lambda t, eid, act: (eid[t], 0, 0)),
            pl.BlockSpec((1, F, H), lambda t, eid, act: (eid[t], 0, 0)),
            pl.BlockSpec((1, 1, H), lambda t, eid, act: (eid[t], 0, 0)),
        ],
        out_specs=pl.BlockSpec((TM, H), lambda t, eid, act: (t, 0)),
        scratch_shapes=[pltpu.VMEM((2 * F, H), jnp.float32)],
    )
    return pl.pallas_call(
        _gmm_body,
        grid_spec=grid_spec,
        out_shape=jax.ShapeDtypeStruct((NPAD, H), jnp.float32),
    )(tile_eid, tile_act, xs_pad, gate_up_w, gate_up_b, down_w, down_b)


def _comb_body(hid_ref, t1_ref, t2_ref, w_ref, out_ref):
    w = w_ref[...]
    out_ref[...] = (hid_ref[...] + w[:, 0:1] * t1_ref[...]
                    + w[:, 1:2] * t2_ref[...])


def _comb_call(hidden, top1, top2, w):
    return pl.pallas_call(
        _comb_body,
        grid=(NR,),
        in_specs=[
            pl.BlockSpec((BR, H), lambda i: (i, 0)),
            pl.BlockSpec((BR, H), lambda i: (i, 0)),
            pl.BlockSpec((BR, H), lambda i: (i, 0)),
            pl.BlockSpec((BR, K), lambda i: (i, 0)),
        ],
        out_specs=pl.BlockSpec((BR, H), lambda i: (i, 0)),
        out_shape=jax.ShapeDtypeStruct((S, H), jnp.float32),
    )(hidden, top1, top2, w)


def _routing(oh1, oh2):
    oh = jnp.stack([oh1, oh2], axis=1).reshape(S * K, E)
    incl = jnp.cumsum(oh, axis=0)
    counts = incl[-1]
    rank = jnp.sum((incl - oh) * oh, axis=1)
    padded = ((counts + TM - 1) // TM) * TM
    ends_pad = jnp.cumsum(padded)
    padoff = ends_pad - padded
    pos_flat = (jnp.sum(oh * padoff[None, :], axis=1) + rank).astype(jnp.int32)
    tstart = jnp.arange(NT, dtype=jnp.int32)[:, None] * TM
    tile_eid = jnp.sum((tstart >= ends_pad[None, :]).astype(jnp.int32), axis=1)
    tile_eid_c = jnp.minimum(tile_eid, E - 1)
    toh = (tile_eid_c[:, None] == jnp.arange(E, dtype=jnp.int32)[None, :]
           ).astype(jnp.int32)
    tile_act = (((tstart[:, 0] - jnp.sum(toh * padoff[None, :], axis=1))
                 < jnp.sum(toh * counts[None, :], axis=1))
                & (tile_eid < E)).astype(jnp.int32)
    return pos_flat, tile_eid_c, tile_act


def kernel(hidden_states, positions, ln1_w, wq, wk, wv, wo, sinks, ln2_w,
           router_w, router_b, gate_up_w, gate_up_b, down_w, down_b):
    x = hidden_states
    wqkv = jnp.concatenate([wq, wk, wv], axis=1)
    inv = 1.0 / (THETA ** (jnp.arange(HALF, dtype=jnp.float32) / HALF))
    ang = positions.astype(jnp.float32)[:, None] * inv[None, :]
    cos = jnp.cos(ang)
    sin = jnp.sin(ang)

    q, k, v = _qkv_call(x, ln1_w.reshape(1, H), wqkv, cos, sin)
    attn = _attn_call(q, k, v, sinks)
    hidden, h2, oh1, oh2, w = _post_call(attn, wo.reshape(NH, HD, H), x,
                                         ln2_w.reshape(1, H),
                                         router_w, router_b.reshape(1, E))

    pos_flat, tile_eid, tile_act = _routing(oh1, oh2)

    src_slot = (jnp.arange(S * K, dtype=jnp.int32) // K)
    xs_pad = _make_sc_dispatch(H, S * K, NPAD, 32)(h2, src_slot, pos_flat)
    rows = _gmm_call(tile_eid, tile_act, xs_pad, gate_up_w,
                     gate_up_b.reshape(E, 1, 2 * F), down_w,
                     down_b.reshape(E, 1, H))
    pos2 = pos_flat.reshape(S, K)
    top1, top2 = _make_sc_return(H, S, 32)(rows, pos2[:, 0], pos2[:, 1])
    return _comb_call(hidden, top1, top2, w)


# speedup vs baseline: 1.5602x; 1.0116x over previous
import functools

import jax
import jax.numpy as jnp
from jax import lax
from jax.experimental import pallas as pl
from jax.experimental.pallas import tpu as pltpu
from jax.experimental.pallas import tpu_sc as plsc

S = 2048
H = 1024
NH = 16
NKV = 8
HD = 64
E = 8
K = 2
F = 1024
EPS = 1e-5
THETA = 150000.0
ALPHA = 1.702
LIMIT = 7.0
HALF = HD // 2

BQ = 1024
BK = 1024
NQ = S // BQ
NJ = S // BK
BR = 512
NR = S // BR
TM = 256
NT = (S * K) // TM + E
NPAD = NT * TM


def _qkv_body(x_ref, ln_ref, w_ref, cos_ref, sin_ref, q_ref, k_ref, v_ref):
    x = x_ref[...]
    var = jnp.mean(x * x, axis=-1, keepdims=True)
    xn = (x * lax.rsqrt(var + EPS)) * ln_ref[...]
    qkv = jnp.dot(xn, w_ref[...], preferred_element_type=jnp.float32)
    cos = cos_ref[...]
    sin = sin_ref[...]

    def rope_head(b):
        x1 = qkv[:, b:b + HALF]
        x2 = qkv[:, b + HALF:b + HD]
        return jnp.concatenate([x1 * cos - x2 * sin, x1 * sin + x2 * cos],
                               axis=1)

    for h in range(NH):
        q_ref[h] = rope_head(h * HD)
    for h in range(NKV):
        k_ref[h] = rope_head((NH + h) * HD)
        v_ref[h] = qkv[:, (NH + NKV + h) * HD:(NH + NKV + h + 1) * HD]


def _qkv_call(x, ln1_w, wqkv, cos, sin):
    return pl.pallas_call(
        _qkv_body,
        grid=(NR,),
        in_specs=[
            pl.BlockSpec((BR, H), lambda i: (i, 0)),
            pl.BlockSpec((1, H), lambda i: (0, 0)),
            pl.BlockSpec((H, (NH + 2 * NKV) * HD), lambda i: (0, 0)),
            pl.BlockSpec((BR, HALF), lambda i: (i, 0)),
            pl.BlockSpec((BR, HALF), lambda i: (i, 0)),
        ],
        out_specs=[
            pl.BlockSpec((NH, BR, HD), lambda i: (0, i, 0)),
            pl.BlockSpec((NKV, BR, HD), lambda i: (0, i, 0)),
            pl.BlockSpec((NKV, BR, HD), lambda i: (0, i, 0)),
        ],
        out_shape=[
            jax.ShapeDtypeStruct((NH, S, HD), jnp.float32),
            jax.ShapeDtypeStruct((NKV, S, HD), jnp.float32),
            jax.ShapeDtypeStruct((NKV, S, HD), jnp.float32),
        ],
    )(x, ln1_w, wqkv, cos, sin)


def _attn_body(sink_ref, q_ref, k_ref, v_ref, o_ref, m_ref, l_ref, acc_ref):
    h = pl.program_id(0)
    i = pl.program_id(1)
    j = pl.program_id(2)

    @pl.when(j == 0)
    def _init():
        m_ref[...] = jnp.full((BQ, 1), -1e30, jnp.float32)
        l_ref[...] = jnp.zeros((BQ, 1), jnp.float32)
        acc_ref[...] = jnp.zeros((BQ, HD), jnp.float32)

    @pl.when(j * BK < (i + 1) * BQ)
    def _compute():
        q = q_ref[0]
        k = k_ref[0]
        s = lax.dot_general(q, k, (((1,), (1,)), ((), ())),
                            preferred_element_type=jnp.float32)
        s = s * (HD ** -0.5)
        rows = i * BQ + lax.broadcasted_iota(jnp.int32, (BQ, BK), 0)
        cols = j * BK + lax.broadcasted_iota(jnp.int32, (BQ, BK), 1)
        s = jnp.where(rows >= cols, s, -1e30)
        m_prev = m_ref[...]
        m_cur = jnp.maximum(m_prev, jnp.max(s, axis=1, keepdims=True))
        alpha = jnp.exp(m_prev - m_cur)
        p = jnp.exp(s - m_cur)
        l_ref[...] = l_ref[...] * alpha + jnp.sum(p, axis=1, keepdims=True)
        acc_ref[...] = acc_ref[...] * alpha + jnp.dot(
            p, v_ref[0], preferred_element_type=jnp.float32)
        m_ref[...] = m_cur

    @pl.when(j == NJ - 1)
    def _fin():
        sink = sink_ref[h]
        l = l_ref[...] + jnp.exp(sink - m_ref[...])
        o_ref[0] = acc_ref[...] / l


def _attn_call(q, k, v, sinks):
    return pl.pallas_call(
        _attn_body,
        grid=(NH, NQ, NJ),
        in_specs=[
            pl.BlockSpec(memory_space=pltpu.SMEM),
            pl.BlockSpec((1, BQ, HD), lambda h, i, j: (h, i, 0)),
            pl.BlockSpec((1, BK, HD),
                         lambda h, i, j: (h // 2,
                                          jnp.minimum(j, (i * BQ + BQ - 1) // BK),
                                          0)),
            pl.BlockSpec((1, BK, HD),
                         lambda h, i, j: (h // 2,
                                          jnp.minimum(j, (i * BQ + BQ - 1) // BK),
                                          0)),
        ],
        out_specs=pl.BlockSpec((1, BQ, HD), lambda h, i, j: (h, i, 0)),
        out_shape=jax.ShapeDtypeStruct((NH, S, HD), jnp.float32),
        scratch_shapes=[
            pltpu.VMEM((BQ, 1), jnp.float32),
            pltpu.VMEM((BQ, 1), jnp.float32),
            pltpu.VMEM((BQ, HD), jnp.float32),
        ],
    )(sinks, q, k, v)


def _post_body(a_ref, wo_ref, hs_ref, ln_ref, rw_ref, rb_ref,
               hid_ref, h2_ref, oh1_ref, oh2_ref, w_ref):
    att = jnp.dot(a_ref[0], wo_ref[0], preferred_element_type=jnp.float32)
    for h in range(1, NH):
        att = att + jnp.dot(a_ref[h], wo_ref[h],
                            preferred_element_type=jnp.float32)
    hid = att + hs_ref[...]
    hid_ref[...] = hid
    var = jnp.mean(hid * hid, axis=-1, keepdims=True)
    h2 = (hid * lax.rsqrt(var + EPS)) * ln_ref[...]
    h2_ref[...] = h2
    lg = jnp.dot(h2, rw_ref[...],
                 preferred_element_type=jnp.float32) + rb_ref[...]
    iota_e = lax.broadcasted_iota(jnp.int32, (BR, E), 1)
    m1 = jnp.max(lg, axis=1, keepdims=True)
    i1 = jnp.min(jnp.where(lg >= m1, iota_e, E), axis=1, keepdims=True)
    oh1 = iota_e == i1
    lg2 = jnp.where(oh1, -1e30, lg)
    m2 = jnp.max(lg2, axis=1, keepdims=True)
    i2 = jnp.min(jnp.where(lg2 >= m2, iota_e, E), axis=1, keepdims=True)
    oh2 = iota_e == i2
    oh1_ref[...] = oh1.astype(jnp.int32)
    oh2_ref[...] = oh2.astype(jnp.int32)
    w1 = jax.nn.sigmoid(m1 - m2)
    w_ref[...] = jnp.concatenate([w1, 1.0 - w1], axis=1)


def _post_call(attn, wo, hs, ln2_w, router_w, router_b):
    return pl.pallas_call(
        _post_body,
        grid=(NR,),
        in_specs=[
            pl.BlockSpec((NH, BR, HD), lambda i: (0, i, 0)),
            pl.BlockSpec((NH, HD, H), lambda i: (0, 0, 0)),
            pl.BlockSpec((BR, H), lambda i: (i, 0)),
            pl.BlockSpec((1, H), lambda i: (0, 0)),
            pl.BlockSpec((H, E), lambda i: (0, 0)),
            pl.BlockSpec((1, E), lambda i: (0, 0)),
        ],
        out_specs=[
            pl.BlockSpec((BR, H), lambda i: (i, 0)),
            pl.BlockSpec((BR, H), lambda i: (i, 0)),
            pl.BlockSpec((BR, E), lambda i: (i, 0)),
            pl.BlockSpec((BR, E), lambda i: (i, 0)),
            pl.BlockSpec((BR, K), lambda i: (i, 0)),
        ],
        out_shape=[
            jax.ShapeDtypeStruct((S, H), jnp.float32),
            jax.ShapeDtypeStruct((S, H), jnp.float32),
            jax.ShapeDtypeStruct((S, E), jnp.int32),
            jax.ShapeDtypeStruct((S, E), jnp.int32),
            jax.ShapeDtypeStruct((S, K), jnp.float32),
        ],
    )(attn, wo, hs, ln2_w, router_w, router_b)


@functools.lru_cache(maxsize=None)
def _make_sc_return(d, n_rows, chunk):
    info = plsc.get_sparse_core_info()
    nc, ns = info.num_cores, info.num_subcores
    nw = nc * ns
    b_per_w = n_rows // nw
    nch = b_per_w // chunk
    mesh = plsc.VectorSubcoreMesh(core_axis_name="c", subcore_axis_name="s")

    @functools.partial(
        pl.kernel,
        mesh=mesh,
        out_type=[
            jax.ShapeDtypeStruct((n_rows, d), jnp.float32),
            jax.ShapeDtypeStruct((n_rows, d), jnp.float32),
        ],
        scratch_types=[
            pltpu.VMEM((chunk,), jnp.int32),
            pltpu.VMEM((chunk,), jnp.int32),
            pltpu.VMEM((chunk, d), jnp.float32),
            pltpu.VMEM((chunk, d), jnp.float32),
            pltpu.SemaphoreType.DMA,
            pltpu.SemaphoreType.DMA,
        ],
    )
    def gather(table_hbm, idx1_hbm, idx2_hbm, out1_hbm, out2_hbm,
               i1_v, i2_v, r1_v, r2_v, sem1, sem2):
        wid = lax.axis_index("s") * nc + lax.axis_index("c")
        base = wid * b_per_w

        def body(c, carry):
            off = base + c * chunk
            pltpu.sync_copy(idx1_hbm.at[pl.ds(off, chunk)], i1_v)
            pltpu.sync_copy(idx2_hbm.at[pl.ds(off, chunk)], i2_v)
            c1 = pltpu.async_copy(table_hbm.at[i1_v], r1_v, sem1)
            c2 = pltpu.async_copy(table_hbm.at[i2_v], r2_v, sem2)
            c1.wait()
            c2.wait()
            pltpu.sync_copy(r1_v, out1_hbm.at[pl.ds(off, chunk)])
            pltpu.sync_copy(r2_v, out2_hbm.at[pl.ds(off, chunk)])
            return carry

        lax.fori_loop(0, nch, body, 0)

    return gather


@functools.lru_cache(maxsize=None)
def _make_sc_dispatch(d, n_slots, n_out, chunk):
    info = plsc.get_sparse_core_info()
    nc, ns = info.num_cores, info.num_subcores
    nw = nc * ns
    spw = n_slots // nw
    nch = spw // chunk
    mesh = plsc.VectorSubcoreMesh(core_axis_name="c", subcore_axis_name="s")

    @functools.partial(
        pl.kernel,
        mesh=mesh,
        out_type=jax.ShapeDtypeStruct((n_out, d), jnp.float32),
        scratch_types=[
            pltpu.VMEM((chunk,), jnp.int32),
            pltpu.VMEM((chunk,), jnp.int32),
            pltpu.VMEM((chunk, d), jnp.float32),
            pltpu.SemaphoreType.DMA,
        ],
    )
    def dispatch(table_hbm, sidx_hbm, didx_hbm, out_hbm,
                 sidx_v, didx_v, rows_v, sem):
        wid = lax.axis_index("s") * nc + lax.axis_index("c")
        base = wid * spw

        def body(c, carry):
            off = base + c * chunk
            pltpu.sync_copy(sidx_hbm.at[pl.ds(off, chunk)], sidx_v)
            pltpu.sync_copy(didx_hbm.at[pl.ds(off, chunk)], didx_v)
            pltpu.async_copy(table_hbm.at[sidx_v], rows_v, sem).wait()
            pltpu.async_copy(rows_v, out_hbm.at[didx_v], sem).wait()
            return carry

        lax.fori_loop(0, nch, body, 0)

    return dispatch


def _gmm_body(eid_ref, act_ref, xs_ref, guw_ref, gub_ref,
              dw_ref, db_ref, out_ref, dz_ref):
    t = pl.program_id(0)

    @pl.when(act_ref[t] == 1)
    def _():
        changed = (t == 0) | (eid_ref[t] != eid_ref[jnp.maximum(t - 1, 0)])

        @pl.when(changed)
        def _rebuild():
            dz_ref[...] = jnp.repeat(dw_ref[0], 2, axis=0)

        x = xs_ref[...]
        gu = jnp.dot(x, guw_ref[0],
                     preferred_element_type=jnp.float32) + gub_ref[0]
        g = jnp.minimum(gu, LIMIT)
        glu = g * jax.nn.sigmoid(g * ALPHA)
        u1 = jnp.clip(gu, -LIMIT, LIMIT) + 1.0
        lane = lax.broadcasted_iota(jnp.int32, (TM, 2 * F), 1)
        hh = jnp.where(lane % 2 == 0, glu * pltpu.roll(u1, 2 * F - 1, 1),
                       0.0)
        out_ref[...] = jnp.dot(hh, dz_ref[...],
                               preferred_element_type=jnp.float32) + db_ref[0]


def _gmm_call(tile_eid, tile_act, xs_pad, gate_up_w, gate_up_b,
              down_w, down_b):
    grid_spec = pltpu.PrefetchScalarGridSpec(
        num_scalar_prefetch=2,
        grid=(NT,),
        in_specs=[
            pl.BlockSpec((TM, H), lambda t, eid, act: (t, 0)),
            pl.BlockSpec((1, H, 2 * F), lambda t, eid, act: (eid[t], 0, 0)),
            pl.BlockSpec((1, 1, 2 * F), lambda t, eid, act: (eid[t], 0, 0)),
            pl.BlockSpec((1, F, H), lambda t, eid, act: (eid[t], 0, 0)),
            pl.BlockSpec((1, 1, H), lambda t, eid, act: (eid[t], 0, 0)),
        ],
        out_specs=pl.BlockSpec((TM, H), lambda t, eid, act: (t, 0)),
        scratch_shapes=[pltpu.VMEM((2 * F, H), jnp.float32)],
    )
    return pl.pallas_call(
        _gmm_body,
        grid_spec=grid_spec,
        out_shape=jax.ShapeDtypeStruct((NPAD, H), jnp.float32),
    )(tile_eid, tile_act, xs_pad, gate_up_w, gate_up_b, down_w, down_b)


def _comb_body(hid_ref, t1_ref, t2_ref, w_ref, out_ref):
    w = w_ref[...]
    out_ref[...] = (hid_ref[...] + w[:, 0:1] * t1_ref[...]
                    + w[:, 1:2] * t2_ref[...])


def _comb_call(hidden, top1, top2, w):
    return pl.pallas_call(
        _comb_body,
        grid=(NR,),
        in_specs=[
            pl.BlockSpec((BR, H), lambda i: (i, 0)),
            pl.BlockSpec((BR, H), lambda i: (i, 0)),
            pl.BlockSpec((BR, H), lambda i: (i, 0)),
            pl.BlockSpec((BR, K), lambda i: (i, 0)),
        ],
        out_specs=pl.BlockSpec((BR, H), lambda i: (i, 0)),
        out_shape=jax.ShapeDtypeStruct((S, H), jnp.float32),
    )(hidden, top1, top2, w)


def _routing(oh1, oh2):
    oh = jnp.stack([oh1, oh2], axis=1).reshape(S * K, E)
    incl = jnp.cumsum(oh, axis=0)
    counts = incl[-1]
    rank = jnp.sum((incl - oh) * oh, axis=1)
    padded = ((counts + TM - 1) // TM) * TM
    ends_pad = jnp.cumsum(padded)
    padoff = ends_pad - padded
    pos_flat = (jnp.sum(oh * padoff[None, :], axis=1) + rank).astype(jnp.int32)
    tstart = jnp.arange(NT, dtype=jnp.int32)[:, None] * TM
    tile_eid = jnp.sum((tstart >= ends_pad[None, :]).astype(jnp.int32), axis=1)
    tile_eid_c = jnp.minimum(tile_eid, E - 1)
    toh = (tile_eid_c[:, None] == jnp.arange(E, dtype=jnp.int32)[None, :]
           ).astype(jnp.int32)
    tile_act = (((tstart[:, 0] - jnp.sum(toh * padoff[None, :], axis=1))
                 < jnp.sum(toh * counts[None, :], axis=1))
                & (tile_eid < E)).astype(jnp.int32)
    return pos_flat, tile_eid_c, tile_act


def kernel(hidden_states, positions, ln1_w, wq, wk, wv, wo, sinks, ln2_w,
           router_w, router_b, gate_up_w, gate_up_b, down_w, down_b):
    x = hidden_states
    wqkv = jnp.concatenate([wq, wk, wv], axis=1)
    inv = 1.0 / (THETA ** (jnp.arange(HALF, dtype=jnp.float32) / HALF))
    ang = positions.astype(jnp.float32)[:, None] * inv[None, :]
    cos = jnp.cos(ang)
    sin = jnp.sin(ang)

    q, k, v = _qkv_call(x, ln1_w.reshape(1, H), wqkv, cos, sin)
    attn = _attn_call(q, k, v, sinks)
    hidden, h2, oh1, oh2, w = _post_call(attn, wo.reshape(NH, HD, H), x,
                                         ln2_w.reshape(1, H),
                                         router_w, router_b.reshape(1, E))

    pos_flat, tile_eid, tile_act = _routing(oh1, oh2)

    src_slot = (jnp.arange(S * K, dtype=jnp.int32) // K)
    xs_pad = _make_sc_dispatch(H, S * K, NPAD, 32)(h2, src_slot, pos_flat)
    rows = _gmm_call(tile_eid, tile_act, xs_pad, gate_up_w,
                     gate_up_b.reshape(E, 1, 2 * F), down_w,
                     down_b.reshape(E, 1, H))
    pos2 = pos_flat.reshape(S, K)
    top1, top2 = _make_sc_return(H, S, 32)(rows, pos2[:, 0], pos2[:, 1])
    return _comb_call(hidden, top1, top2, w)


# speedup vs baseline: 1.5730x; 1.0082x over previous
import functools

import jax
import jax.numpy as jnp
from jax import lax
from jax.experimental import pallas as pl
from jax.experimental.pallas import tpu as pltpu
from jax.experimental.pallas import tpu_sc as plsc

S = 2048
H = 1024
NH = 16
NKV = 8
HD = 64
E = 8
K = 2
F = 1024
EPS = 1e-5
THETA = 150000.0
ALPHA = 1.702
LIMIT = 7.0
HALF = HD // 2

BQ = 1024
BK = 1024
NQ = S // BQ
NJ = S // BK
BR = 512
NR = S // BR
TM = 256
NT = (S * K) // TM + E
NPAD = NT * TM


def _qkv_body(x_ref, ln_ref, w_ref, cos_ref, sin_ref, q_ref, k_ref, v_ref):
    x = x_ref[...]
    var = jnp.mean(x * x, axis=-1, keepdims=True)
    xn = (x * lax.rsqrt(var + EPS)) * ln_ref[...]
    qkv = jnp.dot(xn, w_ref[...], preferred_element_type=jnp.float32)
    cos = cos_ref[...]
    sin = sin_ref[...]

    def rope_head(b):
        x1 = qkv[:, b:b + HALF]
        x2 = qkv[:, b + HALF:b + HD]
        return jnp.concatenate([x1 * cos - x2 * sin, x1 * sin + x2 * cos],
                               axis=1)

    for h in range(NH):
        q_ref[h] = rope_head(h * HD)
    for h in range(NKV):
        k_ref[h] = rope_head((NH + h) * HD)
        v_ref[h] = qkv[:, (NH + NKV + h) * HD:(NH + NKV + h + 1) * HD]


def _qkv_call(x, ln1_w, wqkv, cos, sin):
    return pl.pallas_call(
        _qkv_body,
        grid=(NR,),
        in_specs=[
            pl.BlockSpec((BR, H), lambda i: (i, 0)),
            pl.BlockSpec((1, H), lambda i: (0, 0)),
            pl.BlockSpec((H, (NH + 2 * NKV) * HD), lambda i: (0, 0)),
            pl.BlockSpec((BR, HALF), lambda i: (i, 0)),
            pl.BlockSpec((BR, HALF), lambda i: (i, 0)),
        ],
        out_specs=[
            pl.BlockSpec((NH, BR, HD), lambda i: (0, i, 0)),
            pl.BlockSpec((NKV, BR, HD), lambda i: (0, i, 0)),
            pl.BlockSpec((NKV, BR, HD), lambda i: (0, i, 0)),
        ],
        out_shape=[
            jax.ShapeDtypeStruct((NH, S, HD), jnp.float32),
            jax.ShapeDtypeStruct((NKV, S, HD), jnp.float32),
            jax.ShapeDtypeStruct((NKV, S, HD), jnp.float32),
        ],
    )(x, ln1_w, wqkv, cos, sin)


def _attn_body(sink_ref, q_ref, k_ref, v_ref, o_ref, m_ref, l_ref, acc_ref):
    h = pl.program_id(0)
    i = pl.program_id(1)
    j = pl.program_id(2)

    @pl.when(j == 0)
    def _init():
        m_ref[...] = jnp.full((BQ, 1), -1e30, jnp.float32)
        l_ref[...] = jnp.zeros((BQ, 1), jnp.float32)
        acc_ref[...] = jnp.zeros((BQ, HD), jnp.float32)

    @pl.when(j * BK < (i + 1) * BQ)
    def _compute():
        q = q_ref[0]
        k = k_ref[0]
        s = lax.dot_general(q, k, (((1,), (1,)), ((), ())),
                            preferred_element_type=jnp.float32)
        s = s * (HD ** -0.5)
        rows = i * BQ + lax.broadcasted_iota(jnp.int32, (BQ, BK), 0)
        cols = j * BK + lax.broadcasted_iota(jnp.int32, (BQ, BK), 1)
        s = jnp.where(rows >= cols, s, -1e30)
        m_prev = m_ref[...]
        m_cur = jnp.maximum(m_prev, jnp.max(s, axis=1, keepdims=True))
        alpha = jnp.exp(m_prev - m_cur)
        p = jnp.exp(s - m_cur)
        l_ref[...] = l_ref[...] * alpha + jnp.sum(p, axis=1, keepdims=True)
        acc_ref[...] = acc_ref[...] * alpha + jnp.dot(
            p, v_ref[0], preferred_element_type=jnp.float32)
        m_ref[...] = m_cur

    @pl.when(j == NJ - 1)
    def _fin():
        sink = sink_ref[h]
        l = l_ref[...] + jnp.exp(sink - m_ref[...])
        o_ref[0] = acc_ref[...] / l


def _attn_call(q, k, v, sinks):
    return pl.pallas_call(
        _attn_body,
        grid=(NH, NQ, NJ),
        in_specs=[
            pl.BlockSpec(memory_space=pltpu.SMEM),
            pl.BlockSpec((1, BQ, HD), lambda h, i, j: (h, i, 0)),
            pl.BlockSpec((1, BK, HD),
                         lambda h, i, j: (h // 2,
                                          jnp.minimum(j, (i * BQ + BQ - 1) // BK),
                                          0)),
            pl.BlockSpec((1, BK, HD),
                         lambda h, i, j: (h // 2,
                                          jnp.minimum(j, (i * BQ + BQ - 1) // BK),
                                          0)),
        ],
        out_specs=pl.BlockSpec((1, BQ, HD), lambda h, i, j: (h, i, 0)),
        out_shape=jax.ShapeDtypeStruct((NH, S, HD), jnp.float32),
        scratch_shapes=[
            pltpu.VMEM((BQ, 1), jnp.float32),
            pltpu.VMEM((BQ, 1), jnp.float32),
            pltpu.VMEM((BQ, HD), jnp.float32),
        ],
    )(sinks, q, k, v)


def _post_body(a_ref, wo_ref, hs_ref, ln_ref, rw_ref, rb_ref,
               hid_ref, h2_ref, oh1_ref, oh2_ref, w_ref):
    att = jnp.dot(a_ref[0], wo_ref[0], preferred_element_type=jnp.float32)
    for h in range(1, NH):
        att = att + jnp.dot(a_ref[h], wo_ref[h],
                            preferred_element_type=jnp.float32)
    hid = att + hs_ref[...]
    hid_ref[...] = hid
    var = jnp.mean(hid * hid, axis=-1, keepdims=True)
    h2 = (hid * lax.rsqrt(var + EPS)) * ln_ref[...]
    h2_ref[...] = h2
    lg = jnp.dot(h2, rw_ref[...],
                 preferred_element_type=jnp.float32) + rb_ref[...]
    iota_e = lax.broadcasted_iota(jnp.int32, (BR, E), 1)
    m1 = jnp.max(lg, axis=1, keepdims=True)
    i1 = jnp.min(jnp.where(lg >= m1, iota_e, E), axis=1, keepdims=True)
    oh1 = iota_e == i1
    lg2 = jnp.where(oh1, -1e30, lg)
    m2 = jnp.max(lg2, axis=1, keepdims=True)
    i2 = jnp.min(jnp.where(lg2 >= m2, iota_e, E), axis=1, keepdims=True)
    oh2 = iota_e == i2
    oh1_ref[...] = oh1.astype(jnp.int32)
    oh2_ref[...] = oh2.astype(jnp.int32)
    w1 = jax.nn.sigmoid(m1 - m2)
    w_ref[...] = jnp.concatenate([w1, 1.0 - w1], axis=1)


def _post_call(attn, wo, hs, ln2_w, router_w, router_b):
    return pl.pallas_call(
        _post_body,
        grid=(NR,),
        in_specs=[
            pl.BlockSpec((NH, BR, HD), lambda i: (0, i, 0)),
            pl.BlockSpec((NH, HD, H), lambda i: (0, 0, 0)),
            pl.BlockSpec((BR, H), lambda i: (i, 0)),
            pl.BlockSpec((1, H), lambda i: (0, 0)),
            pl.BlockSpec((H, E), lambda i: (0, 0)),
            pl.BlockSpec((1, E), lambda i: (0, 0)),
        ],
        out_specs=[
            pl.BlockSpec((BR, H), lambda i: (i, 0)),
            pl.BlockSpec((BR, H), lambda i: (i, 0)),
            pl.BlockSpec((BR, E), lambda i: (i, 0)),
            pl.BlockSpec((BR, E), lambda i: (i, 0)),
            pl.BlockSpec((BR, K), lambda i: (i, 0)),
        ],
        out_shape=[
            jax.ShapeDtypeStruct((S, H), jnp.float32),
            jax.ShapeDtypeStruct((S, H), jnp.float32),
            jax.ShapeDtypeStruct((S, E), jnp.int32),
            jax.ShapeDtypeStruct((S, E), jnp.int32),
            jax.ShapeDtypeStruct((S, K), jnp.float32),
        ],
    )(attn, wo, hs, ln2_w, router_w, router_b)


@functools.lru_cache(maxsize=None)
def _make_sc_return(d, n_rows, chunk):
    info = plsc.get_sparse_core_info()
    nc, ns = info.num_cores, info.num_subcores
    nw = nc * ns
    b_per_w = n_rows // nw
    nch = b_per_w // chunk
    mesh = plsc.VectorSubcoreMesh(core_axis_name="c", subcore_axis_name="s")

    @functools.partial(
        pl.kernel,
        mesh=mesh,
        out_type=[
            jax.ShapeDtypeStruct((n_rows, d), jnp.float32),
            jax.ShapeDtypeStruct((n_rows, d), jnp.float32),
        ],
        scratch_types=[
            pltpu.VMEM((chunk,), jnp.int32),
            pltpu.VMEM((chunk,), jnp.int32),
            pltpu.VMEM((chunk, d), jnp.float32),
            pltpu.VMEM((chunk, d), jnp.float32),
            pltpu.SemaphoreType.DMA,
            pltpu.SemaphoreType.DMA,
        ],
    )
    def gather(table_hbm, idx1_hbm, idx2_hbm, out1_hbm, out2_hbm,
               i1_v, i2_v, r1_v, r2_v, sem1, sem2):
        wid = lax.axis_index("s") * nc + lax.axis_index("c")
        base = wid * b_per_w

        def body(c, carry):
            off = base + c * chunk
            pltpu.sync_copy(idx1_hbm.at[pl.ds(off, chunk)], i1_v)
            pltpu.sync_copy(idx2_hbm.at[pl.ds(off, chunk)], i2_v)
            c1 = pltpu.async_copy(table_hbm.at[i1_v], r1_v, sem1)
            c2 = pltpu.async_copy(table_hbm.at[i2_v], r2_v, sem2)
            c1.wait()
            c2.wait()
            pltpu.sync_copy(r1_v, out1_hbm.at[pl.ds(off, chunk)])
            pltpu.sync_copy(r2_v, out2_hbm.at[pl.ds(off, chunk)])
            return carry

        lax.fori_loop(0, nch, body, 0)

    return gather


@functools.lru_cache(maxsize=None)
def _make_sc_dispatch(d, n_slots, n_out, chunk):
    info = plsc.get_sparse_core_info()
    nc, ns = info.num_cores, info.num_subcores
    nw = nc * ns
    spw = n_slots // nw
    nch = spw // chunk
    mesh = plsc.VectorSubcoreMesh(core_axis_name="c", subcore_axis_name="s")

    @functools.partial(
        pl.kernel,
        mesh=mesh,
        out_type=jax.ShapeDtypeStruct((n_out, d), jnp.float32),
        scratch_types=[
            pltpu.VMEM((chunk,), jnp.int32),
            pltpu.VMEM((chunk,), jnp.int32),
            pltpu.VMEM((chunk, d), jnp.float32),
            pltpu.SemaphoreType.DMA,
        ],
    )
    def dispatch(table_hbm, sidx_hbm, didx_hbm, out_hbm,
                 sidx_v, didx_v, rows_v, sem):
        wid = lax.axis_index("s") * nc + lax.axis_index("c")
        base = wid * spw

        def body(c, carry):
            off = base + c * chunk
            pltpu.sync_copy(sidx_hbm.at[pl.ds(off, chunk)], sidx_v)
            pltpu.sync_copy(didx_hbm.at[pl.ds(off, chunk)], didx_v)
            pltpu.async_copy(table_hbm.at[sidx_v], rows_v, sem).wait()
            pltpu.async_copy(rows_v, out_hbm.at[didx_v], sem).wait()
            return carry

        lax.fori_loop(0, nch, body, 0)

    return dispatch


def _gmm_body(eid_ref, act_ref, xs_ref, guw_ref, gub_ref,
              dw_ref, db_ref, out_ref, dz_ref):
    t = pl.program_id(0)

    @pl.when(act_ref[t] == 1)
    def _():
        changed = (t == 0) | (eid_ref[t] != eid_ref[jnp.maximum(t - 1, 0)])

        @pl.when(changed)
        def _rebuild():
            dz_ref[...] = jnp.repeat(dw_ref[0], 2, axis=0)

        x = xs_ref[...]
        gu = jnp.dot(x, guw_ref[0],
                     preferred_element_type=jnp.float32) + gub_ref[0]
        g = jnp.minimum(gu, LIMIT)
        glu = g * jax.nn.sigmoid(g * ALPHA)
        u1 = jnp.clip(gu, -LIMIT, LIMIT) + 1.0
        lane = lax.broadcasted_iota(jnp.int32, (TM, 2 * F), 1)
        hh = jnp.where(lane % 2 == 0, glu * pltpu.roll(u1, 2 * F - 1, 1),
                       0.0)
        out_ref[...] = jnp.dot(hh, dz_ref[...],
                               preferred_element_type=jnp.float32) + db_ref[0]


def _gmm_call(tile_eid, tile_act, xs_pad, gate_up_w, gate_up_b,
              down_w, down_b):
    grid_spec = pltpu.PrefetchScalarGridSpec(
        num_scalar_prefetch=2,
        grid=(NT,),
        in_specs=[
            pl.BlockSpec((TM, H), lambda t, eid, act: (t, 0)),
            pl.BlockSpec((1, H, 2 * F), lambda t, eid, act: (eid[t], 0, 0)),
            pl.BlockSpec((1, 1, 2 * F), lambda t, eid, act: (eid[t], 0, 0)),
            pl.BlockSpec((1, F, H), lambda t, eid, act: (eid[t], 0, 0)),
            pl.BlockSpec((1, 1, H), lambda t, eid, act: (eid[t], 0, 0)),
        ],
        out_specs=pl.BlockSpec((TM, H), lambda t, eid, act: (t, 0)),
        scratch_shapes=[pltpu.VMEM((2 * F, H), jnp.float32)],
    )
    return pl.pallas_call(
        _gmm_body,
        grid_spec=grid_spec,
        out_shape=jax.ShapeDtypeStruct((NPAD, H), jnp.float32),
    )(tile_eid, tile_act, xs_pad, gate_up_w, gate_up_b, down_w, down_b)


def _comb_body(hid_ref, t1_ref, t2_ref, w_ref, out_ref):
    w = w_ref[...]
    out_ref[...] = (hid_ref[...] + w[:, 0:1] * t1_ref[...]
                    + w[:, 1:2] * t2_ref[...])


def _comb_call(hidden, top1, top2, w):
    return pl.pallas_call(
        _comb_body,
        grid=(NR,),
        in_specs=[
            pl.BlockSpec((BR, H), lambda i: (i, 0)),
            pl.BlockSpec((BR, H), lambda i: (i, 0)),
            pl.BlockSpec((BR, H), lambda i: (i, 0)),
            pl.BlockSpec((BR, K), lambda i: (i, 0)),
        ],
        out_specs=pl.BlockSpec((BR, H), lambda i: (i, 0)),
        out_shape=jax.ShapeDtypeStruct((S, H), jnp.float32),
    )(hidden, top1, top2, w)


def _routing(oh1, oh2):
    oh = jnp.stack([oh1, oh2], axis=1).reshape(S * K, E)
    incl = jnp.cumsum(oh, axis=0)
    counts = incl[-1]
    rank = jnp.sum((incl - oh) * oh, axis=1)
    padded = ((counts + TM - 1) // TM) * TM
    ends_pad = jnp.cumsum(padded)
    padoff = ends_pad - padded
    pos_flat = (jnp.sum(oh * padoff[None, :], axis=1) + rank).astype(jnp.int32)
    tstart = jnp.arange(NT, dtype=jnp.int32)[:, None] * TM
    tile_eid = jnp.sum((tstart >= ends_pad[None, :]).astype(jnp.int32), axis=1)
    tile_eid_c = jnp.minimum(tile_eid, E - 1)
    toh = (tile_eid_c[:, None] == jnp.arange(E, dtype=jnp.int32)[None, :]
           ).astype(jnp.int32)
    tile_act = (((tstart[:, 0] - jnp.sum(toh * padoff[None, :], axis=1))
                 < jnp.sum(toh * counts[None, :], axis=1))
                & (tile_eid < E)).astype(jnp.int32)
    return pos_flat, tile_eid_c, tile_act


def kernel(hidden_states, positions, ln1_w, wq, wk, wv, wo, sinks, ln2_w,
           router_w, router_b, gate_up_w, gate_up_b, down_w, down_b):
    x = hidden_states
    wqkv = jnp.concatenate([wq, wk, wv], axis=1)
    inv = 1.0 / (THETA ** (jnp.arange(HALF, dtype=jnp.float32) / HALF))
    ang = positions.astype(jnp.float32)[:, None] * inv[None, :]
    cos = jnp.cos(ang)
    sin = jnp.sin(ang)

    q, k, v = _qkv_call(x, ln1_w.reshape(1, H), wqkv, cos, sin)
    attn = _attn_call(q, k, v, sinks)
    hidden, h2, oh1, oh2, w = _post_call(attn, wo.reshape(NH, HD, H), x,
                                         ln2_w.reshape(1, H),
                                         router_w, router_b.reshape(1, E))

    pos_flat, tile_eid, tile_act = _routing(oh1, oh2)

    src_slot = (jnp.arange(S * K, dtype=jnp.int32) // K)
    xs_pad = _make_sc_dispatch(H, S * K, NPAD, 64)(h2, src_slot, pos_flat)
    rows = _gmm_call(tile_eid, tile_act, xs_pad, gate_up_w,
                     gate_up_b.reshape(E, 1, 2 * F), down_w,
                     down_b.reshape(E, 1, H))
    pos2 = pos_flat.reshape(S, K)
    top1, top2 = _make_sc_return(H, S, 32)(rows, pos2[:, 0], pos2[:, 1])
    return _comb_call(hidden, top1, top2, w)
